# baseline (speedup 1.0000x reference)
import functools

import jax
import jax.numpy as jnp
from jax import lax
from jax.experimental import pallas as pl
from jax.experimental.pallas import tpu as pltpu
from jax.experimental.pallas import tpu_sc as plsc

N_H = 16384
B = 64
HB = B // 2
N_IN = 512
N_OUT = 256
E = 268435
NT = 16
C = 128
CHUNKS = 136
E_PAD = NT * CHUNKS * C
CAP3 = 1152
CAP2 = 2048
ROWS_PER_TILE = N_H // NT
WB = 128
LANES = 16
GPC = C // LANES
JV = HB // LANES


def _splat_i32(x):
    return jnp.full((LANES,), x, jnp.int32)


def _body(hs0, srcs, dsts, vals, zer, out, hs_a, hs_b,
          acc, c_src_all, cnt_all,
          src_t, dst_t, val_t, c_src, c_dst, c_val,
          c2_src, c2_dst, c2_val, flag, tmp, cntb, cntb_all, rows16,
          r0, r1, g0, g1, s0, s1):
    rows = [r0, r1]
    semg = [g0, g1]
    sems = [s0, s1]
    cid = lax.axis_index("c")
    tid = lax.axis_index("s")
    zi = jnp.zeros((LANES,), jnp.int32)
    zf = jnp.zeros((LANES,), jnp.float32)
    ones16 = jnp.ones((LANES,), jnp.int32)

    pltpu.sync_copy(srcs.at[:, tid], src_t)
    pltpu.sync_copy(dsts.at[:, tid], dst_t)
    pltpu.sync_copy(vals.at[:, tid], val_t)

    def z3_body(i, carry):
        c_src[pl.ds(i * LANES, LANES)] = zi
        c_dst[pl.ds(i * LANES, LANES)] = zi
        c_val[pl.ds(i * LANES, LANES)] = zf
        return carry

    lax.fori_loop(0, CAP3 // LANES, z3_body, 0)

    def z2_body(i, carry):
        c2_src[pl.ds(i * LANES, LANES)] = zi
        c2_dst[pl.ds(i * LANES, LANES)] = zi
        c2_val[pl.ds(i * LANES, LANES)] = zf
        return carry

    lax.fori_loop(0, CAP2 // LANES, z2_body, 0)

    def cp_body(g, cnt):
        c = g // GPC
        off = (g % GPC) * LANES
        d = dst_t[c, pl.ds(off, LANES)]
        m = d >= N_H - N_OUT
        npop = jnp.sum(m.astype(jnp.int32))
        ok = (cnt + npop) <= CAP3

        @pl.when(jnp.logical_and(ok, npop > 0))
        def _():
            s = src_t[c, pl.ds(off, LANES)]
            v = val_t[c, pl.ds(off, LANES)]
            plsc.store_compressed(c_dst.at[pl.ds(cnt, LANES)], d, mask=m)
            plsc.store_compressed(c_src.at[pl.ds(cnt, LANES)], s, mask=m)
            plsc.store_compressed(c_val.at[pl.ds(cnt, LANES)], v, mask=m)

        return jnp.where(ok, cnt + npop, jnp.int32(CAP3 + 1))

    n3 = lax.fori_loop(0, CHUNKS * GPC, cp_body, jnp.int32(0))

    pltpu.sync_copy(c_src, c_src_all.at[tid])
    cntb[pl.ds(0, LANES)] = _splat_i32(n3)
    pltpu.sync_copy(cntb.at[pl.ds(0, 8)], cnt_all.at[pl.ds(tid * 8, 8)])
    plsc.subcore_barrier()
    pltpu.sync_copy(cnt_all, cntb_all)
    mx = cntb_all[pl.ds(0, LANES)]
    for i in range(1, NT * 8 // LANES):
        mx = jnp.maximum(mx, cntb_all[pl.ds(i * LANES, LANES)])
    ovf = jnp.max(mx) > CAP3

    def zf_body(i, carry):
        flag[pl.ds(i * LANES, LANES)] = zi
        return carry

    lax.fori_loop(0, N_H // LANES, zf_body, 0)

    @pl.when(jnp.logical_not(ovf))
    def _():
        for r in range(NT):
            pltpu.sync_copy(c_src_all.at[r], tmp)

            def fb(g, carry):
                svec = tmp[pl.ds(g * LANES, LANES)]
                plsc.store_scatter(flag, [svec], ones16)
                return carry

            lax.fori_loop(0, CAP3 // LANES, fb, 0)

    def scale(b, c, vref):
        def scale_body(i, carry2):
            if vref is val_t:
                vv = vref[c, pl.ds(i * LANES, LANES)]
            else:
                vv = vref[pl.ds(c * C + i * LANES, LANES)]
            for u in range(LANES):
                e = i * LANES + u
                vs = jnp.take_along_axis(
                    vv, jnp.full((LANES,), u, jnp.int32), axis=0)
                for j in range(JV):
                    sl = (e, pl.ds(j * LANES, LANES))
                    rows[b][sl] = rows[b][sl] * vs
            return carry2

        lax.fori_loop(0, GPC, scale_body, 0)

    ins = [hs0, hs_a, hs_b]
    outs = [hs_a, hs_b, None]
    for step in range(3):
        hs_in = ins[step].at[cid]

        def full_body(c, carry):
            pltpu.async_copy(hs_in.at[src_t.at[c]], rows[0], semg[0]).wait()
            scale(0, c, val_t)
            pltpu.sync_copy(rows[0], acc.at[dst_t.at[c]], add=True)
            return carry

        pltpu.sync_copy(zer, acc.at[pl.ds(tid * ROWS_PER_TILE, ROWS_PER_TILE)])
        plsc.subcore_barrier()

        if step == 0:
            def chunk_body(c, carry):
                smin = jnp.min(src_t[c, pl.ds(0, LANES)])

                @pl.when(smin < N_IN)
                def _():
                    full_body(c, 0)

                return carry

            lax.fori_loop(0, CHUNKS, chunk_body, 0)
        elif step == 1:
            def flush(cnt2):
                nch = (cnt2 + C - 1) // C

                def start_gather(b, j):
                    pltpu.async_copy(
                        hs_in.at[c2_src.at[pl.ds(j * C, C)]], rows[b],
                        semg[b])

                def proc(b, j):
                    pltpu.make_async_copy(
                        hs_in.at[c2_src.at[pl.ds(j * C, C)]], rows[b],
                        semg[b]).wait()

                    @pl.when(j + 1 < nch)
                    def _():
                        start_gather(1 - b, j + 1)

                    scale(b, j, c2_val)
                    for u in range(GPC):
                        dvec = c2_dst[pl.ds(j * C + u * LANES, LANES)]
                        pltpu.async_copy(
                            rows[b].at[pl.ds(u * LANES, LANES)],
                            acc.at[dvec], sems[b], add=True)
                    for u in range(GPC):
                        pltpu.make_async_copy(
                            rows[b].at[pl.ds(u * LANES, LANES)],
                            acc.at[zi], sems[b]).wait()

                @pl.when(nch > 0)
                def _():
                    start_gather(0, jnp.int32(0))

                def fl_body(i, carry):
                    for b in range(2):
                        j = i * 2 + b

                        @pl.when(j < nch)
                        def _(b=b, j=j):
                            proc(b, j)

                    return carry

                lax.fori_loop(0, CAP2 // C // 2, fl_body, 0)

                lax.fori_loop(0, CAP2 // LANES, z2_body, 0)

            @pl.when(jnp.logical_not(ovf))
            def _():
                def ft_body(g, cnt2):
                    c = g // GPC
                    off = (g % GPC) * LANES
                    d = dst_t[c, pl.ds(off, LANES)]
                    fv = plsc.load_gather(flag, [d])
                    m = fv > 0
                    npop = jnp.sum(m.astype(jnp.int32))
                    do_flush = (cnt2 + npop) > CAP2

                    @pl.when(do_flush)
                    def _():
                        flush(cnt2)

                    base = jnp.where(do_flush, jnp.int32(0), cnt2)

                    @pl.when(npop > 0)
                    def _():
                        s = src_t[c, pl.ds(off, LANES)]
                        v = val_t[c, pl.ds(off, LANES)]
                        plsc.store_compressed(
                            c2_dst.at[pl.ds(base, LANES)], d, mask=m)
                        plsc.store_compressed(
                            c2_src.at[pl.ds(base, LANES)], s, mask=m)
                        plsc.store_compressed(
                            c2_val.at[pl.ds(base, LANES)], v, mask=m)

                    return base + npop

                cnt2 = lax.fori_loop(0, CHUNKS * GPC, ft_body, jnp.int32(0))
                flush(cnt2)

            @pl.when(ovf)
            def _():
                lax.fori_loop(0, CHUNKS, full_body, 0)
        else:
            @pl.when(n3 <= CAP3)
            def _():
                ng = (n3 + LANES - 1) // LANES

                def g_body(g, carry):
                    svec = c_src[pl.ds(g * LANES, LANES)]
                    dvec = c_dst[pl.ds(g * LANES, LANES)]
                    vvec = c_val[pl.ds(g * LANES, LANES)]
                    pltpu.async_copy(hs_in.at[svec], rows16, semg[0]).wait()
                    for u in range(LANES):
                        vs = jnp.take_along_axis(
                            vvec, jnp.full((LANES,), u, jnp.int32), axis=0)
                        for j in range(JV):
                            sl = (u, pl.ds(j * LANES, LANES))
                            rows16[sl] = rows16[sl] * vs
                    pltpu.sync_copy(rows16, acc.at[dvec], add=True)
                    return carry

                lax.fori_loop(0, ng, g_body, 0)

            @pl.when(n3 > CAP3)
            def _():
                lax.fori_loop(0, CHUNKS, full_body, 0)

        plsc.subcore_barrier()

        if step < 2:
            hs_out = outs[step].at[cid]
            base = tid * ROWS_PER_TILE

            def wb_body(k, carry):
                rbase = base + k * WB
                pltpu.sync_copy(acc.at[pl.ds(rbase, WB)], rows[0])

                def relu_body(r4, carry2):
                    for u in range(4):
                        r = r4 * 4 + u
                        keep = _splat_i32(rbase + r) >= (N_H - N_OUT)
                        for j in range(JV):
                            sl = (r, pl.ds(j * LANES, LANES))
                            x = rows[0][sl]
                            rows[0][sl] = jnp.where(keep, x,
                                                    jnp.maximum(x, 0.0))
                    return carry2

                lax.fori_loop(0, WB // 4, relu_body, 0)
                pltpu.sync_copy(rows[0], hs_out.at[pl.ds(rbase, WB)])
                return carry

            lax.fori_loop(0, ROWS_PER_TILE // WB, wb_body, 0)
        else:
            @pl.when(tid == NT - 1)
            def _():
                for k in range(N_OUT // WB):
                    rbase = N_H - N_OUT + k * WB
                    pltpu.sync_copy(acc.at[pl.ds(rbase, WB)], rows[0])

                    def sig_body(r4, carry2):
                        for u in range(4):
                            r = r4 * 4 + u
                            for j in range(JV):
                                sl = (r, pl.ds(j * LANES, LANES))
                                x = rows[0][sl]
                                rows[0][sl] = 1.0 / (1.0 + jnp.exp(-x))
                        return carry2

                    lax.fori_loop(0, WB // 4, sig_body, 0)
                    pltpu.sync_copy(rows[0], out.at[cid].at[pl.ds(k * WB, WB)])


_sun_sc = functools.partial(
    pl.kernel,
    out_type=(
        jax.ShapeDtypeStruct((2, N_OUT, HB), jnp.float32),
        jax.ShapeDtypeStruct((2, N_H, HB), jnp.float32),
        jax.ShapeDtypeStruct((2, N_H, HB), jnp.float32),
    ),
    mesh=plsc.VectorSubcoreMesh(core_axis_name="c", subcore_axis_name="s"),
    compiler_params=pltpu.CompilerParams(
        needs_layout_passes=False, use_tc_tiling_on_sc=False
    ),
    scratch_types=[
        pltpu.VMEM_SHARED((N_H, HB), jnp.float32),
        pltpu.VMEM_SHARED((NT, CAP3), jnp.int32),
        pltpu.VMEM_SHARED((NT * 8,), jnp.int32),
        pltpu.VMEM((CHUNKS, C), jnp.int32),
        pltpu.VMEM((CHUNKS, C), jnp.int32),
        pltpu.VMEM((CHUNKS, C), jnp.float32),
        pltpu.VMEM((CAP3,), jnp.int32),
        pltpu.VMEM((CAP3,), jnp.int32),
        pltpu.VMEM((CAP3,), jnp.float32),
        pltpu.VMEM((CAP2,), jnp.int32),
        pltpu.VMEM((CAP2,), jnp.int32),
        pltpu.VMEM((CAP2,), jnp.float32),
        pltpu.VMEM((N_H,), jnp.int32),
        pltpu.VMEM((CAP3,), jnp.int32),
        pltpu.VMEM((LANES,), jnp.int32),
        pltpu.VMEM((NT * 8,), jnp.int32),
        pltpu.VMEM((LANES, HB), jnp.float32),
        pltpu.VMEM((C, HB), jnp.float32),
        pltpu.VMEM((C, HB), jnp.float32),
        pltpu.SemaphoreType.DMA,
        pltpu.SemaphoreType.DMA,
        pltpu.SemaphoreType.DMA,
        pltpu.SemaphoreType.DMA,
    ],
)(_body)


@jax.jit
def kernel(inp, edge_indices, edge_values):
    src = edge_indices[0].astype(jnp.int32)
    dst = edge_indices[1].astype(jnp.int32)
    val = edge_values.astype(jnp.float32)
    pad = E_PAD - E
    src = jnp.pad(src, (0, pad)).reshape(CHUNKS, NT, C)
    dst = jnp.pad(dst, (0, pad)).reshape(CHUNKS, NT, C)
    val = jnp.pad(val, (0, pad)).reshape(CHUNKS, NT, C)
    hs0 = jnp.zeros((N_H, B), jnp.float32).at[:N_IN].set(inp.T)
    hs0 = hs0.reshape(N_H, 2, HB).transpose(1, 0, 2)
    zer = jnp.zeros((ROWS_PER_TILE, HB), jnp.float32)
    out, _, _ = _sun_sc(hs0, src, dst, val, zer)
    return jnp.concatenate([out[0], out[1]], axis=1).T


# speedup vs baseline: 1.0002x; 1.0002x over previous
import functools

import jax
import jax.numpy as jnp
from jax import lax
from jax.experimental import pallas as pl
from jax.experimental.pallas import tpu as pltpu
from jax.experimental.pallas import tpu_sc as plsc

N_H = 16384
B = 64
HB = B // 2
N_IN = 512
N_OUT = 256
E = 268435
NT = 16
C = 128
CHUNKS = 136
E_PAD = NT * CHUNKS * C
CAP3 = 1152
CAP2 = 2048
ROWS_PER_TILE = N_H // NT
WB = 128
LANES = 16
GPC = C // LANES
JV = HB // LANES


def _splat_i32(x):
    return jnp.full((LANES,), x, jnp.int32)


def _body(hs0, srcs, dsts, vals, zer, out, hs_a, hs_b,
          acc, c_src_all, cnt_all,
          src_t, dst_t, val_t, c_src, c_dst, c_val,
          c2_src, c2_dst, c2_val, flag, tmp, cntb, cntb_all, rows16,
          r0, r1, g0, g1, s0, s1):
    rows = [r0, r1]
    semg = [g0, g1]
    sems = [s0, s1]
    cid = lax.axis_index("c")
    tid = lax.axis_index("s")
    zi = jnp.zeros((LANES,), jnp.int32)
    zf = jnp.zeros((LANES,), jnp.float32)
    ones16 = jnp.ones((LANES,), jnp.int32)

    _ns = jax.named_scope
    pltpu.sync_copy(srcs.at[:, tid], src_t)
    pltpu.sync_copy(dsts.at[:, tid], dst_t)
    pltpu.sync_copy(vals.at[:, tid], val_t)

    def z3_body(i, carry):
        c_src[pl.ds(i * LANES, LANES)] = zi
        c_dst[pl.ds(i * LANES, LANES)] = zi
        c_val[pl.ds(i * LANES, LANES)] = zf
        return carry

    lax.fori_loop(0, CAP3 // LANES, z3_body, 0)

    def z2_body(i, carry):
        c2_src[pl.ds(i * LANES, LANES)] = zi
        c2_dst[pl.ds(i * LANES, LANES)] = zi
        c2_val[pl.ds(i * LANES, LANES)] = zf
        return carry

    lax.fori_loop(0, CAP2 // LANES, z2_body, 0)

    def cp_body(g, cnt):
        c = g // GPC
        off = (g % GPC) * LANES
        d = dst_t[c, pl.ds(off, LANES)]
        m = d >= N_H - N_OUT
        npop = jnp.sum(m.astype(jnp.int32))
        ok = (cnt + npop) <= CAP3

        @pl.when(jnp.logical_and(ok, npop > 0))
        def _():
            s = src_t[c, pl.ds(off, LANES)]
            v = val_t[c, pl.ds(off, LANES)]
            plsc.store_compressed(c_dst.at[pl.ds(cnt, LANES)], d, mask=m)
            plsc.store_compressed(c_src.at[pl.ds(cnt, LANES)], s, mask=m)
            plsc.store_compressed(c_val.at[pl.ds(cnt, LANES)], v, mask=m)

        return jnp.where(ok, cnt + npop, jnp.int32(CAP3 + 1))

    with _ns("compact3"):
        n3 = lax.fori_loop(0, CHUNKS * GPC, cp_body, jnp.int32(0))

    pltpu.sync_copy(c_src, c_src_all.at[tid])
    cntb[pl.ds(0, LANES)] = _splat_i32(n3)
    pltpu.sync_copy(cntb.at[pl.ds(0, 8)], cnt_all.at[pl.ds(tid * 8, 8)])
    plsc.subcore_barrier()
    pltpu.sync_copy(cnt_all, cntb_all)
    mx = cntb_all[pl.ds(0, LANES)]
    for i in range(1, NT * 8 // LANES):
        mx = jnp.maximum(mx, cntb_all[pl.ds(i * LANES, LANES)])
    ovf = jnp.max(mx) > CAP3

    def zf_body(i, carry):
        flag[pl.ds(i * LANES, LANES)] = zi
        return carry

    lax.fori_loop(0, N_H // LANES, zf_body, 0)

    @pl.when(jnp.logical_not(ovf))
    def _():
      with _ns("flagmap"):
        for r in range(NT):
            pltpu.sync_copy(c_src_all.at[r], tmp)

            def fb(g, carry):
                svec = tmp[pl.ds(g * LANES, LANES)]
                plsc.store_scatter(flag, [svec], ones16)
                return carry

            lax.fori_loop(0, CAP3 // LANES, fb, 0)

    def scale(b, c, vref):
        def scale_body(i, carry2):
            if vref is val_t:
                vv = vref[c, pl.ds(i * LANES, LANES)]
            else:
                vv = vref[pl.ds(c * C + i * LANES, LANES)]
            for u in range(LANES):
                e = i * LANES + u
                vs = jnp.take_along_axis(
                    vv, jnp.full((LANES,), u, jnp.int32), axis=0)
                for j in range(JV):
                    sl = (e, pl.ds(j * LANES, LANES))
                    rows[b][sl] = rows[b][sl] * vs
            return carry2

        lax.fori_loop(0, GPC, scale_body, 0)

    ins = [hs0, hs_a, hs_b]
    outs = [hs_a, hs_b, None]
    for step in range(3):
        hs_in = ins[step].at[cid]

        def full_body(c, carry):
            pltpu.async_copy(hs_in.at[src_t.at[c]], rows[0], semg[0]).wait()
            scale(0, c, val_t)
            pltpu.sync_copy(rows[0], acc.at[dst_t.at[c]], add=True)
            return carry

        pltpu.sync_copy(zer, acc.at[pl.ds(tid * ROWS_PER_TILE, ROWS_PER_TILE)])
        plsc.subcore_barrier()

        if step == 0:
            def chunk_body(c, carry):
                smin = jnp.min(src_t[c, pl.ds(0, LANES)])

                @pl.when(smin < N_IN)
                def _():
                    full_body(c, 0)

                return carry

            with _ns("scatter0"):
                lax.fori_loop(0, CHUNKS, chunk_body, 0)
        elif step == 1:
            def flush(cnt2):
                nch = (cnt2 + C - 1) // C

                def start_gather(b, j):
                    pltpu.async_copy(
                        hs_in.at[c2_src.at[pl.ds(j * C, C)]], rows[b],
                        semg[b])

                def proc(b, j):
                    pltpu.make_async_copy(
                        hs_in.at[c2_src.at[pl.ds(j * C, C)]], rows[b],
                        semg[b]).wait()

                    @pl.when(j + 1 < nch)
                    def _():
                        start_gather(1 - b, j + 1)

                    scale(b, j, c2_val)
                    for u in range(GPC):
                        dvec = c2_dst[pl.ds(j * C + u * LANES, LANES)]
                        pltpu.async_copy(
                            rows[b].at[pl.ds(u * LANES, LANES)],
                            acc.at[dvec], sems[b], add=True)
                    for u in range(GPC):
                        pltpu.make_async_copy(
                            rows[b].at[pl.ds(u * LANES, LANES)],
                            acc.at[zi], sems[b]).wait()

                @pl.when(nch > 0)
                def _():
                    start_gather(0, jnp.int32(0))

                def fl_body(i, carry):
                    for b in range(2):
                        j = i * 2 + b

                        @pl.when(j < nch)
                        def _(b=b, j=j):
                            proc(b, j)

                    return carry

                lax.fori_loop(0, CAP2 // C // 2, fl_body, 0)

                lax.fori_loop(0, CAP2 // LANES, z2_body, 0)

            @pl.when(jnp.logical_not(ovf))
            def _():
                def ft_body(g, cnt2):
                    c = g // GPC
                    off = (g % GPC) * LANES
                    d = dst_t[c, pl.ds(off, LANES)]
                    fv = plsc.load_gather(flag, [d])
                    m = fv > 0
                    npop = jnp.sum(m.astype(jnp.int32))
                    do_flush = (cnt2 + npop) > CAP2

                    @pl.when(do_flush)
                    def _():
                        flush(cnt2)

                    base = jnp.where(do_flush, jnp.int32(0), cnt2)

                    @pl.when(npop > 0)
                    def _():
                        s = src_t[c, pl.ds(off, LANES)]
                        v = val_t[c, pl.ds(off, LANES)]
                        plsc.store_compressed(
                            c2_dst.at[pl.ds(base, LANES)], d, mask=m)
                        plsc.store_compressed(
                            c2_src.at[pl.ds(base, LANES)], s, mask=m)
                        plsc.store_compressed(
                            c2_val.at[pl.ds(base, LANES)], v, mask=m)

                    return base + npop

                with _ns("scatter1"):
                    cnt2 = lax.fori_loop(0, CHUNKS * GPC, ft_body,
                                         jnp.int32(0))
                    flush(cnt2)

            @pl.when(ovf)
            def _():
                lax.fori_loop(0, CHUNKS, full_body, 0)
        else:
            @pl.when(n3 <= CAP3)
            def _():
                ng = (n3 + LANES - 1) // LANES

                def g_body(g, carry):
                    svec = c_src[pl.ds(g * LANES, LANES)]
                    dvec = c_dst[pl.ds(g * LANES, LANES)]
                    vvec = c_val[pl.ds(g * LANES, LANES)]
                    pltpu.async_copy(hs_in.at[svec], rows16, semg[0]).wait()
                    for u in range(LANES):
                        vs = jnp.take_along_axis(
                            vvec, jnp.full((LANES,), u, jnp.int32), axis=0)
                        for j in range(JV):
                            sl = (u, pl.ds(j * LANES, LANES))
                            rows16[sl] = rows16[sl] * vs
                    pltpu.sync_copy(rows16, acc.at[dvec], add=True)
                    return carry

                with _ns("scatter2"):
                    lax.fori_loop(0, ng, g_body, 0)

            @pl.when(n3 > CAP3)
            def _():
                lax.fori_loop(0, CHUNKS, full_body, 0)

        plsc.subcore_barrier()

        if step < 2:
            hs_out = outs[step].at[cid]
            base = tid * ROWS_PER_TILE

            def wb_body(k, carry):
                rbase = base + k * WB
                pltpu.sync_copy(acc.at[pl.ds(rbase, WB)], rows[0])

                def relu_body(r4, carry2):
                    for u in range(4):
                        r = r4 * 4 + u
                        keep = _splat_i32(rbase + r) >= (N_H - N_OUT)
                        for j in range(JV):
                            sl = (r, pl.ds(j * LANES, LANES))
                            x = rows[0][sl]
                            rows[0][sl] = jnp.where(keep, x,
                                                    jnp.maximum(x, 0.0))
                    return carry2

                lax.fori_loop(0, WB // 4, relu_body, 0)
                pltpu.sync_copy(rows[0], hs_out.at[pl.ds(rbase, WB)])
                return carry

            with _ns("wb" + str(step)):
                lax.fori_loop(0, ROWS_PER_TILE // WB, wb_body, 0)
        else:
            @pl.when(tid == NT - 1)
            def _():
                for k in range(N_OUT // WB):
                    rbase = N_H - N_OUT + k * WB
                    pltpu.sync_copy(acc.at[pl.ds(rbase, WB)], rows[0])

                    def sig_body(r4, carry2):
                        for u in range(4):
                            r = r4 * 4 + u
                            for j in range(JV):
                                sl = (r, pl.ds(j * LANES, LANES))
                                x = rows[0][sl]
                                rows[0][sl] = 1.0 / (1.0 + jnp.exp(-x))
                        return carry2

                    lax.fori_loop(0, WB // 4, sig_body, 0)
                    pltpu.sync_copy(rows[0], out.at[cid].at[pl.ds(k * WB, WB)])


_sun_sc = functools.partial(
    pl.kernel,
    out_type=(
        jax.ShapeDtypeStruct((2, N_OUT, HB), jnp.float32),
        jax.ShapeDtypeStruct((2, N_H, HB), jnp.float32),
        jax.ShapeDtypeStruct((2, N_H, HB), jnp.float32),
    ),
    mesh=plsc.VectorSubcoreMesh(core_axis_name="c", subcore_axis_name="s"),
    compiler_params=pltpu.CompilerParams(
        needs_layout_passes=False, use_tc_tiling_on_sc=False
    ),
    scratch_types=[
        pltpu.VMEM_SHARED((N_H, HB), jnp.float32),
        pltpu.VMEM_SHARED((NT, CAP3), jnp.int32),
        pltpu.VMEM_SHARED((NT * 8,), jnp.int32),
        pltpu.VMEM((CHUNKS, C), jnp.int32),
        pltpu.VMEM((CHUNKS, C), jnp.int32),
        pltpu.VMEM((CHUNKS, C), jnp.float32),
        pltpu.VMEM((CAP3,), jnp.int32),
        pltpu.VMEM((CAP3,), jnp.int32),
        pltpu.VMEM((CAP3,), jnp.float32),
        pltpu.VMEM((CAP2,), jnp.int32),
        pltpu.VMEM((CAP2,), jnp.int32),
        pltpu.VMEM((CAP2,), jnp.float32),
        pltpu.VMEM((N_H,), jnp.int32),
        pltpu.VMEM((CAP3,), jnp.int32),
        pltpu.VMEM((LANES,), jnp.int32),
        pltpu.VMEM((NT * 8,), jnp.int32),
        pltpu.VMEM((LANES, HB), jnp.float32),
        pltpu.VMEM((C, HB), jnp.float32),
        pltpu.VMEM((C, HB), jnp.float32),
        pltpu.SemaphoreType.DMA,
        pltpu.SemaphoreType.DMA,
        pltpu.SemaphoreType.DMA,
        pltpu.SemaphoreType.DMA,
    ],
)(_body)


@jax.jit
def kernel(inp, edge_indices, edge_values):
    src = edge_indices[0].astype(jnp.int32)
    dst = edge_indices[1].astype(jnp.int32)
    val = edge_values.astype(jnp.float32)
    pad = E_PAD - E
    src = jnp.pad(src, (0, pad)).reshape(CHUNKS, NT, C)
    dst = jnp.pad(dst, (0, pad)).reshape(CHUNKS, NT, C)
    val = jnp.pad(val, (0, pad)).reshape(CHUNKS, NT, C)
    hs0 = jnp.zeros((N_H, B), jnp.float32).at[:N_IN].set(inp.T)
    hs0 = hs0.reshape(N_H, 2, HB).transpose(1, 0, 2)
    zer = jnp.zeros((ROWS_PER_TILE, HB), jnp.float32)
    out, _, _ = _sun_sc(hs0, src, dst, val, zer)
    return jnp.concatenate([out[0], out[1]], axis=1).T


# speedup vs baseline: 1.3714x; 1.3711x over previous
import functools

import jax
import jax.numpy as jnp
from jax import lax
from jax.experimental import pallas as pl
from jax.experimental.pallas import tpu as pltpu
from jax.experimental.pallas import tpu_sc as plsc

N_H = 16384
B = 64
HB = B // 2
N_IN = 512
N_OUT = 256
E = 268435
NT = 16
C = 128
CHUNKS = 136
E_PAD = NT * CHUNKS * C
CAP3 = 1152
CAP2 = 2048
ROWS_PER_TILE = N_H // NT
WB = 128
LANES = 16
GPC = C // LANES
JV = HB // LANES


def _splat_i32(x):
    return jnp.full((LANES,), x, jnp.int32)


def _body(inps, srcs, dsts, vals, zer, out, hs_a, hs_b,
          acc, c_src_all, cnt_all,
          src_t, dst_t, val_t, c_src, c_dst, c_val,
          c2_src, c2_dst, c2_val, fmap, tmp, cntb, cntb_all, rows16,
          r0, r1, g0, g1, s0, s1):
    rows = [r0, r1]
    semg = [g0, g1]
    sems = [s0, s1]
    cid = lax.axis_index("c")
    tid = lax.axis_index("s")
    zi = jnp.zeros((LANES,), jnp.int32)
    zf = jnp.zeros((LANES,), jnp.float32)
    ones16f = jnp.ones((LANES,), jnp.float32)

    _ns = jax.named_scope
    pltpu.sync_copy(srcs.at[:, tid], src_t)
    pltpu.sync_copy(dsts.at[:, tid], dst_t)
    pltpu.sync_copy(vals.at[:, tid], val_t)

    def z3_body(i, carry):
        c_src[pl.ds(i * LANES, LANES)] = zi
        c_dst[pl.ds(i * LANES, LANES)] = zi
        c_val[pl.ds(i * LANES, LANES)] = zf
        return carry

    lax.fori_loop(0, CAP3 // LANES, z3_body, 0)

    def z2_body(i, carry):
        c2_src[pl.ds(i * LANES, LANES)] = zi
        c2_dst[pl.ds(i * LANES, LANES)] = zi
        c2_val[pl.ds(i * LANES, LANES)] = zf
        return carry

    lax.fori_loop(0, CAP2 // LANES, z2_body, 0)

    def cp_body(g, cnt):
        c = g // GPC
        off = (g % GPC) * LANES
        d = dst_t[c, pl.ds(off, LANES)]
        m = d >= N_H - N_OUT
        npop = plsc.all_reduce_population_count(m)[0]
        ok = (cnt + npop) <= CAP3

        @pl.when(jnp.logical_and(ok, npop > 0))
        def _():
            s = src_t[c, pl.ds(off, LANES)]
            v = val_t[c, pl.ds(off, LANES)]
            plsc.store_compressed(c_dst.at[pl.ds(cnt, LANES)], d, mask=m)
            plsc.store_compressed(c_src.at[pl.ds(cnt, LANES)], s, mask=m)
            plsc.store_compressed(c_val.at[pl.ds(cnt, LANES)], v, mask=m)

        return jnp.where(ok, cnt + npop, jnp.int32(CAP3 + 1))

    with _ns("compact3"):
        n3 = lax.fori_loop(0, CHUNKS * GPC, cp_body, jnp.int32(0))

    pltpu.sync_copy(c_src, c_src_all.at[tid])
    cntb[pl.ds(0, LANES)] = _splat_i32(n3)
    pltpu.sync_copy(cntb.at[pl.ds(0, 8)], cnt_all.at[pl.ds(tid * 8, 8)])
    plsc.subcore_barrier()
    pltpu.sync_copy(cnt_all, cntb_all)
    mx = cntb_all[pl.ds(0, LANES)]
    for i in range(1, NT * 8 // LANES):
        mx = jnp.maximum(mx, cntb_all[pl.ds(i * LANES, LANES)])
    ovf = jnp.max(mx) > CAP3

    def scale(b, c, vref):
        def scale_body(i, carry2):
            if vref is val_t:
                vv = vref[c, pl.ds(i * LANES, LANES)]
            else:
                vv = vref[pl.ds(c * C + i * LANES, LANES)]
            for u in range(LANES):
                e = i * LANES + u
                vs = jnp.take_along_axis(
                    vv, jnp.full((LANES,), u, jnp.int32), axis=0)
                for j in range(JV):
                    sl = (e, pl.ds(j * LANES, LANES))
                    rows[b][sl] = rows[b][sl] * vs
            return carry2

        lax.fori_loop(0, GPC, scale_body, 0)

    ins = [None, hs_a, hs_b]
    outs = [hs_a, hs_b, None]
    iota16 = lax.iota(jnp.int32, LANES)
    for step in range(3):
        hs_in = ins[step].at[cid] if step > 0 else None

        def full_body(c, carry):
            pltpu.async_copy(hs_in.at[src_t.at[c]], rows[0], semg[0]).wait()
            scale(0, c, val_t)
            pltpu.sync_copy(rows[0], acc.at[dst_t.at[c]], add=True)
            return carry

        pltpu.sync_copy(zer, acc.at[pl.ds(tid * ROWS_PER_TILE, ROWS_PER_TILE)])
        plsc.subcore_barrier()

        if step == 0:
            pltpu.sync_copy(inps.at[cid], fmap)

            def chunk_body(c, carry):
                smin = jnp.min(src_t[c, pl.ds(0, LANES)])

                @pl.when(smin < N_IN)
                def _():
                    def grp(i, carry2):
                        svec = src_t[c, pl.ds(i * LANES, LANES)]
                        vv = val_t[c, pl.ds(i * LANES, LANES)]
                        for u in range(LANES):
                            uu = jnp.full((LANES,), u, jnp.int32)
                            sv = jnp.take_along_axis(svec, uu, axis=0)
                            vs = jnp.take_along_axis(vv, uu, axis=0)
                            okm = sv < N_IN
                            sv = jnp.where(okm, sv, 0)
                            vs = jnp.where(okm, vs, 0.0)
                            for j in range(JV):
                                addr = sv * HB + (j * LANES + iota16)
                                x = plsc.load_gather(fmap, [addr])
                                sl = (i * LANES + u, pl.ds(j * LANES, LANES))
                                rows[0][sl] = x * vs
                        return carry2

                    lax.fori_loop(0, GPC, grp, 0)
                    pltpu.sync_copy(rows[0], acc.at[dst_t.at[c]], add=True)

                return carry

            with _ns("scatter0"):
                lax.fori_loop(0, CHUNKS, chunk_body, 0)
        elif step == 1:
            with _ns("flagmap"):
                def zf_body(i, carry):
                    fmap[pl.ds(i * LANES, LANES)] = zf
                    return carry

                lax.fori_loop(0, N_H // LANES, zf_body, 0)

                @pl.when(jnp.logical_not(ovf))
                def _():
                    for r in range(NT):
                        pltpu.sync_copy(c_src_all.at[r], tmp)

                        def fb(g, carry):
                            svec = tmp[pl.ds(g * LANES, LANES)]
                            plsc.store_scatter(fmap, [svec], ones16f)
                            return carry

                        lax.fori_loop(0, CAP3 // LANES, fb, 0)

            def flush(cnt2):
                nch = (cnt2 + C - 1) // C

                def start_gather(b, j):
                    pltpu.async_copy(
                        hs_in.at[c2_src.at[pl.ds(j * C, C)]], rows[b],
                        semg[b])

                def proc(b, j):
                    pltpu.make_async_copy(
                        hs_in.at[c2_src.at[pl.ds(j * C, C)]], rows[b],
                        semg[b]).wait()

                    @pl.when(j + 1 < nch)
                    def _():
                        start_gather(1 - b, j + 1)

                    scale(b, j, c2_val)
                    for u in range(GPC):
                        dvec = c2_dst[pl.ds(j * C + u * LANES, LANES)]
                        pltpu.async_copy(
                            rows[b].at[pl.ds(u * LANES, LANES)],
                            acc.at[dvec], sems[b], add=True)
                    for u in range(GPC):
                        pltpu.make_async_copy(
                            rows[b].at[pl.ds(u * LANES, LANES)],
                            acc.at[zi], sems[b]).wait()

                @pl.when(nch > 0)
                def _():
                    start_gather(0, jnp.int32(0))

                def fl_body(i, carry):
                    for b in range(2):
                        j = i * 2 + b

                        @pl.when(j < nch)
                        def _(b=b, j=j):
                            proc(b, j)

                    return carry

                lax.fori_loop(0, CAP2 // C // 2, fl_body, 0)

                lax.fori_loop(0, CAP2 // LANES, z2_body, 0)

            @pl.when(jnp.logical_not(ovf))
            def _():
                def ft_body(g, cnt2):
                    c = g // GPC
                    off = (g % GPC) * LANES
                    d = dst_t[c, pl.ds(off, LANES)]
                    fv = plsc.load_gather(fmap, [d])
                    m = fv > 0.0
                    npop = plsc.all_reduce_population_count(m)[0]
                    do_flush = (cnt2 + npop) > CAP2

                    @pl.when(do_flush)
                    def _():
                        flush(cnt2)

                    base = jnp.where(do_flush, jnp.int32(0), cnt2)
                    s = src_t[c, pl.ds(off, LANES)]
                    v = val_t[c, pl.ds(off, LANES)]
                    plsc.store_compressed(
                        c2_dst.at[pl.ds(base, LANES)], d, mask=m)
                    plsc.store_compressed(
                        c2_src.at[pl.ds(base, LANES)], s, mask=m)
                    plsc.store_compressed(
                        c2_val.at[pl.ds(base, LANES)], v, mask=m)

                    return base + npop

                with _ns("scatter1"):
                    cnt2 = lax.fori_loop(0, CHUNKS * GPC, ft_body,
                                         jnp.int32(0))
                    flush(cnt2)

            @pl.when(ovf)
            def _():
                lax.fori_loop(0, CHUNKS, full_body, 0)
        else:
            @pl.when(n3 <= CAP3)
            def _():
                ng = (n3 + LANES - 1) // LANES

                def g_body(g, carry):
                    svec = c_src[pl.ds(g * LANES, LANES)]
                    dvec = c_dst[pl.ds(g * LANES, LANES)]
                    vvec = c_val[pl.ds(g * LANES, LANES)]
                    pltpu.async_copy(hs_in.at[svec], rows16, semg[0]).wait()
                    for u in range(LANES):
                        vs = jnp.take_along_axis(
                            vvec, jnp.full((LANES,), u, jnp.int32), axis=0)
                        for j in range(JV):
                            sl = (u, pl.ds(j * LANES, LANES))
                            rows16[sl] = rows16[sl] * vs
                    pltpu.sync_copy(rows16, acc.at[dvec], add=True)
                    return carry

                with _ns("scatter2"):
                    lax.fori_loop(0, ng, g_body, 0)

            @pl.when(n3 > CAP3)
            def _():
                lax.fori_loop(0, CHUNKS, full_body, 0)

        plsc.subcore_barrier()

        if step < 2:
            hs_out = outs[step].at[cid]
            base = tid * ROWS_PER_TILE

            def wb_body(k, carry):
                rbase = base + k * WB
                pltpu.sync_copy(acc.at[pl.ds(rbase, WB)], rows[0])

                def relu_body(r4, carry2):
                    for u in range(4):
                        r = r4 * 4 + u
                        keep = _splat_i32(rbase + r) >= (N_H - N_OUT)
                        for j in range(JV):
                            sl = (r, pl.ds(j * LANES, LANES))
                            x = rows[0][sl]
                            rows[0][sl] = jnp.where(keep, x,
                                                    jnp.maximum(x, 0.0))
                    return carry2

                lax.fori_loop(0, WB // 4, relu_body, 0)
                pltpu.sync_copy(rows[0], hs_out.at[pl.ds(rbase, WB)])
                return carry

            with _ns("wb" + str(step)):
                lax.fori_loop(0, ROWS_PER_TILE // WB, wb_body, 0)
        else:
            @pl.when(tid == NT - 1)
            def _():
                for k in range(N_OUT // WB):
                    rbase = N_H - N_OUT + k * WB
                    pltpu.sync_copy(acc.at[pl.ds(rbase, WB)], rows[0])

                    def sig_body(r4, carry2):
                        for u in range(4):
                            r = r4 * 4 + u
                            for j in range(JV):
                                sl = (r, pl.ds(j * LANES, LANES))
                                x = rows[0][sl]
                                rows[0][sl] = 1.0 / (1.0 + jnp.exp(-x))
                        return carry2

                    lax.fori_loop(0, WB // 4, sig_body, 0)
                    pltpu.sync_copy(rows[0], out.at[cid].at[pl.ds(k * WB, WB)])


_sun_sc = functools.partial(
    pl.kernel,
    out_type=(
        jax.ShapeDtypeStruct((2, N_OUT, HB), jnp.float32),
        jax.ShapeDtypeStruct((2, N_H, HB), jnp.float32),
        jax.ShapeDtypeStruct((2, N_H, HB), jnp.float32),
    ),
    mesh=plsc.VectorSubcoreMesh(core_axis_name="c", subcore_axis_name="s"),
    compiler_params=pltpu.CompilerParams(
        needs_layout_passes=False, use_tc_tiling_on_sc=False
    ),
    scratch_types=[
        pltpu.VMEM_SHARED((N_H, HB), jnp.float32),
        pltpu.VMEM_SHARED((NT, CAP3), jnp.int32),
        pltpu.VMEM_SHARED((NT * 8,), jnp.int32),
        pltpu.VMEM((CHUNKS, C), jnp.int32),
        pltpu.VMEM((CHUNKS, C), jnp.int32),
        pltpu.VMEM((CHUNKS, C), jnp.float32),
        pltpu.VMEM((CAP3,), jnp.int32),
        pltpu.VMEM((CAP3,), jnp.int32),
        pltpu.VMEM((CAP3,), jnp.float32),
        pltpu.VMEM((CAP2,), jnp.int32),
        pltpu.VMEM((CAP2,), jnp.int32),
        pltpu.VMEM((CAP2,), jnp.float32),
        pltpu.VMEM((N_H,), jnp.float32),
        pltpu.VMEM((CAP3,), jnp.int32),
        pltpu.VMEM((LANES,), jnp.int32),
        pltpu.VMEM((NT * 8,), jnp.int32),
        pltpu.VMEM((LANES, HB), jnp.float32),
        pltpu.VMEM((C, HB), jnp.float32),
        pltpu.VMEM((C, HB), jnp.float32),
        pltpu.SemaphoreType.DMA,
        pltpu.SemaphoreType.DMA,
        pltpu.SemaphoreType.DMA,
        pltpu.SemaphoreType.DMA,
    ],
)(_body)


@jax.jit
def kernel(inp, edge_indices, edge_values):
    src = edge_indices[0].astype(jnp.int32)
    dst = edge_indices[1].astype(jnp.int32)
    val = edge_values.astype(jnp.float32)
    pad = E_PAD - E
    src = jnp.pad(src, (0, pad)).reshape(CHUNKS, NT, C)
    dst = jnp.pad(dst, (0, pad)).reshape(CHUNKS, NT, C)
    val = jnp.pad(val, (0, pad)).reshape(CHUNKS, NT, C)
    inps = inp.T.reshape(N_IN, 2, HB).transpose(1, 0, 2).reshape(
        2, N_IN * HB)
    zer = jnp.zeros((ROWS_PER_TILE, HB), jnp.float32)
    out, _, _ = _sun_sc(inps, src, dst, val, zer)
    return jnp.concatenate([out[0], out[1]], axis=1).T


# speedup vs baseline: 1.4537x; 1.0600x over previous
import functools

import jax
import jax.numpy as jnp
from jax import lax
from jax.experimental import pallas as pl
from jax.experimental.pallas import tpu as pltpu
from jax.experimental.pallas import tpu_sc as plsc

N_H = 16384
B = 64
HB = B // 2
N_IN = 512
N_OUT = 256
E = 268435
NT = 16
C = 128
CHUNKS = 136
E_PAD = NT * CHUNKS * C
CAP3 = 1152
CAP2 = 4096
ROWS_PER_TILE = N_H // NT
WB = 128
LANES = 16
GPC = C // LANES
FU = 4
JV = HB // LANES


def _splat_i32(x):
    return jnp.full((LANES,), x, jnp.int32)


def _body(inps, srcs, dsts, vals, zer, out, hs_a, hs_b,
          acc, c_src_all, cnt_all,
          src_t, dst_t, val_t, c_src, c_dst, c_val,
          c2_src, c2_dst, c2_val, fmap, tmp, cntb, cntb_all, rows16,
          r0, r1, g0, g1, s0, s1):
    rows = [r0, r1]
    semg = [g0, g1]
    sems = [s0, s1]
    cid = lax.axis_index("c")
    tid = lax.axis_index("s")
    zi = jnp.zeros((LANES,), jnp.int32)
    zf = jnp.zeros((LANES,), jnp.float32)
    ones16f = jnp.ones((LANES,), jnp.float32)

    _ns = jax.named_scope
    pltpu.sync_copy(srcs.at[:, tid], src_t)
    pltpu.sync_copy(dsts.at[:, tid], dst_t)
    pltpu.sync_copy(vals.at[:, tid], val_t)

    def z3_body(i, carry):
        c_src[pl.ds(i * LANES, LANES)] = zi
        c_dst[pl.ds(i * LANES, LANES)] = zi
        c_val[pl.ds(i * LANES, LANES)] = zf
        return carry

    lax.fori_loop(0, CAP3 // LANES, z3_body, 0)

    def z2_body(i, carry):
        c2_src[pl.ds(i * LANES, LANES)] = zi
        c2_dst[pl.ds(i * LANES, LANES)] = zi
        c2_val[pl.ds(i * LANES, LANES)] = zf
        return carry

    lax.fori_loop(0, CAP2 // LANES, z2_body, 0)

    def cp_body(g, cnt):
        c = g // GPC
        off = (g % GPC) * LANES
        d = dst_t[c, pl.ds(off, LANES)]
        m = d >= N_H - N_OUT
        npop = plsc.all_reduce_population_count(m)[0]
        ok = (cnt + npop) <= CAP3

        @pl.when(jnp.logical_and(ok, npop > 0))
        def _():
            s = src_t[c, pl.ds(off, LANES)]
            v = val_t[c, pl.ds(off, LANES)]
            plsc.store_compressed(c_dst.at[pl.ds(cnt, LANES)], d, mask=m)
            plsc.store_compressed(c_src.at[pl.ds(cnt, LANES)], s, mask=m)
            plsc.store_compressed(c_val.at[pl.ds(cnt, LANES)], v, mask=m)

        return jnp.where(ok, cnt + npop, jnp.int32(CAP3 + 1))

    with _ns("compact3"):
        n3 = lax.fori_loop(0, CHUNKS * GPC, cp_body, jnp.int32(0))

    pltpu.sync_copy(c_src, c_src_all.at[tid])
    cntb[pl.ds(0, LANES)] = _splat_i32(n3)
    pltpu.sync_copy(cntb.at[pl.ds(0, 8)], cnt_all.at[pl.ds(tid * 8, 8)])
    plsc.subcore_barrier()
    pltpu.sync_copy(cnt_all, cntb_all)
    mx = cntb_all[pl.ds(0, LANES)]
    for i in range(1, NT * 8 // LANES):
        mx = jnp.maximum(mx, cntb_all[pl.ds(i * LANES, LANES)])
    ovf = jnp.max(mx) > CAP3

    def scale(b, c, vref):
        def scale_body(i, carry2):
            if vref is val_t:
                vv = vref[c, pl.ds(i * LANES, LANES)]
            else:
                vv = vref[pl.ds(c * C + i * LANES, LANES)]
            for u in range(LANES):
                e = i * LANES + u
                vs = jnp.take_along_axis(
                    vv, jnp.full((LANES,), u, jnp.int32), axis=0)
                for j in range(JV):
                    sl = (e, pl.ds(j * LANES, LANES))
                    rows[b][sl] = rows[b][sl] * vs
            return carry2

        lax.fori_loop(0, GPC, scale_body, 0)

    ins = [None, hs_a, hs_b]
    outs = [hs_a, hs_b, None]
    iota16 = lax.iota(jnp.int32, LANES)
    for step in range(3):
        hs_in = ins[step].at[cid] if step > 0 else None

        def full_body(c, carry):
            pltpu.async_copy(hs_in.at[src_t.at[c]], rows[0], semg[0]).wait()
            scale(0, c, val_t)
            pltpu.sync_copy(rows[0], acc.at[dst_t.at[c]], add=True)
            return carry

        pltpu.sync_copy(zer, acc.at[pl.ds(tid * ROWS_PER_TILE, ROWS_PER_TILE)])
        plsc.subcore_barrier()

        if step == 0:
            pltpu.sync_copy(inps.at[cid], fmap)

            def chunk_body(c, carry):
                smin = jnp.min(src_t[c, pl.ds(0, LANES)])

                @pl.when(smin < N_IN)
                def _():
                    def grp(i, carry2):
                        svec = src_t[c, pl.ds(i * LANES, LANES)]
                        vv = val_t[c, pl.ds(i * LANES, LANES)]
                        for u in range(LANES):
                            uu = jnp.full((LANES,), u, jnp.int32)
                            sv = jnp.take_along_axis(svec, uu, axis=0)
                            vs = jnp.take_along_axis(vv, uu, axis=0)
                            okm = sv < N_IN
                            sv = jnp.where(okm, sv, 0)
                            vs = jnp.where(okm, vs, 0.0)
                            for j in range(JV):
                                addr = sv * HB + (j * LANES + iota16)
                                x = plsc.load_gather(fmap, [addr])
                                sl = (i * LANES + u, pl.ds(j * LANES, LANES))
                                rows[0][sl] = x * vs
                        return carry2

                    lax.fori_loop(0, GPC, grp, 0)
                    pltpu.sync_copy(rows[0], acc.at[dst_t.at[c]], add=True)

                return carry

            with _ns("scatter0"):
                lax.fori_loop(0, CHUNKS, chunk_body, 0)
        elif step == 1:
            with _ns("flagmap"):
                def zf_body(i, carry):
                    fmap[pl.ds(i * LANES, LANES)] = zf
                    return carry

                lax.fori_loop(0, N_H // LANES, zf_body, 0)

                @pl.when(jnp.logical_not(ovf))
                def _():
                    for r in range(NT):
                        pltpu.sync_copy(c_src_all.at[r], tmp)

                        def fb(g, carry):
                            svec = tmp[pl.ds(g * LANES, LANES)]
                            plsc.store_scatter(fmap, [svec], ones16f)
                            return carry

                        lax.fori_loop(0, CAP3 // LANES, fb, 0)

            def flush(cnt2):
                nch = (cnt2 + C - 1) // C

                def start_gather(b, j):
                    pltpu.async_copy(
                        hs_in.at[c2_src.at[pl.ds(j * C, C)]], rows[b],
                        semg[b])

                def proc(b, j):
                    pltpu.make_async_copy(
                        hs_in.at[c2_src.at[pl.ds(j * C, C)]], rows[b],
                        semg[b]).wait()

                    @pl.when(j + 1 < nch)
                    def _():
                        start_gather(1 - b, j + 1)

                    scale(b, j, c2_val)
                    for u in range(GPC):
                        dvec = c2_dst[pl.ds(j * C + u * LANES, LANES)]
                        pltpu.async_copy(
                            rows[b].at[pl.ds(u * LANES, LANES)],
                            acc.at[dvec], sems[b], add=True)
                    for u in range(GPC):
                        pltpu.make_async_copy(
                            rows[b].at[pl.ds(u * LANES, LANES)],
                            acc.at[zi], sems[b]).wait()

                @pl.when(nch > 0)
                def _():
                    start_gather(0, jnp.int32(0))

                def fl_body(i, carry):
                    for b in range(2):
                        j = i * 2 + b

                        @pl.when(j < nch)
                        def _(b=b, j=j):
                            proc(b, j)

                    return carry

                lax.fori_loop(0, CAP2 // C // 2, fl_body, 0)

                lax.fori_loop(0, CAP2 // LANES, z2_body, 0)

            @pl.when(jnp.logical_not(ovf))
            def _():
                def ft_body(q, cnt2):
                    ds_, ms_, np_ = [], [], []
                    for k in range(FU):
                        g = q * FU + k
                        c = g // GPC
                        off = (g % GPC) * LANES
                        d = dst_t[c, pl.ds(off, LANES)]
                        fv = plsc.load_gather(fmap, [d])
                        m = fv > 0.0
                        ds_.append(d)
                        ms_.append(m)
                        np_.append(plsc.all_reduce_population_count(m)[0])
                    tot = (np_[0] + np_[1]) + (np_[2] + np_[3])
                    do_flush = (cnt2 + tot) > CAP2

                    @pl.when(do_flush)
                    def _():
                        flush(cnt2)

                    base = jnp.where(do_flush, jnp.int32(0), cnt2)
                    for k in range(FU):
                        g = q * FU + k
                        c = g // GPC
                        off = (g % GPC) * LANES
                        sarr = src_t[c, pl.ds(off, LANES)]
                        varr = val_t[c, pl.ds(off, LANES)]
                        plsc.store_compressed(
                            c2_dst.at[pl.ds(base, LANES)], ds_[k],
                            mask=ms_[k])
                        plsc.store_compressed(
                            c2_src.at[pl.ds(base, LANES)], sarr,
                            mask=ms_[k])
                        plsc.store_compressed(
                            c2_val.at[pl.ds(base, LANES)], varr,
                            mask=ms_[k])
                        base = base + np_[k]
                    return base

                with _ns("scatter1"):
                    cnt2 = lax.fori_loop(0, CHUNKS * GPC // FU, ft_body,
                                         jnp.int32(0))
                    flush(cnt2)

            @pl.when(ovf)
            def _():
                lax.fori_loop(0, CHUNKS, full_body, 0)
        else:
            @pl.when(n3 <= CAP3)
            def _():
                ng = (n3 + LANES - 1) // LANES

                def g_body(g, carry):
                    svec = c_src[pl.ds(g * LANES, LANES)]
                    dvec = c_dst[pl.ds(g * LANES, LANES)]
                    vvec = c_val[pl.ds(g * LANES, LANES)]
                    pltpu.async_copy(hs_in.at[svec], rows16, semg[0]).wait()
                    for u in range(LANES):
                        vs = jnp.take_along_axis(
                            vvec, jnp.full((LANES,), u, jnp.int32), axis=0)
                        for j in range(JV):
                            sl = (u, pl.ds(j * LANES, LANES))
                            rows16[sl] = rows16[sl] * vs
                    pltpu.sync_copy(rows16, acc.at[dvec], add=True)
                    return carry

                with _ns("scatter2"):
                    lax.fori_loop(0, ng, g_body, 0)

            @pl.when(n3 > CAP3)
            def _():
                lax.fori_loop(0, CHUNKS, full_body, 0)

        plsc.subcore_barrier()

        if step < 2:
            hs_out = outs[step].at[cid]
            base = tid * ROWS_PER_TILE

            def wb_body(k, carry):
                rbase = base + k * WB
                pltpu.sync_copy(acc.at[pl.ds(rbase, WB)], rows[0])

                def relu_body(r4, carry2):
                    for u in range(4):
                        r = r4 * 4 + u
                        keep = _splat_i32(rbase + r) >= (N_H - N_OUT)
                        for j in range(JV):
                            sl = (r, pl.ds(j * LANES, LANES))
                            x = rows[0][sl]
                            rows[0][sl] = jnp.where(keep, x,
                                                    jnp.maximum(x, 0.0))
                    return carry2

                lax.fori_loop(0, WB // 4, relu_body, 0)
                pltpu.sync_copy(rows[0], hs_out.at[pl.ds(rbase, WB)])
                return carry

            with _ns("wb" + str(step)):
                lax.fori_loop(0, ROWS_PER_TILE // WB, wb_body, 0)
        else:
            @pl.when(tid == NT - 1)
            def _():
                for k in range(N_OUT // WB):
                    rbase = N_H - N_OUT + k * WB
                    pltpu.sync_copy(acc.at[pl.ds(rbase, WB)], rows[0])

                    def sig_body(r4, carry2):
                        for u in range(4):
                            r = r4 * 4 + u
                            for j in range(JV):
                                sl = (r, pl.ds(j * LANES, LANES))
                                x = rows[0][sl]
                                rows[0][sl] = 1.0 / (1.0 + jnp.exp(-x))
                        return carry2

                    lax.fori_loop(0, WB // 4, sig_body, 0)
                    pltpu.sync_copy(rows[0], out.at[cid].at[pl.ds(k * WB, WB)])


_sun_sc = functools.partial(
    pl.kernel,
    out_type=(
        jax.ShapeDtypeStruct((2, N_OUT, HB), jnp.float32),
        jax.ShapeDtypeStruct((2, N_H, HB), jnp.float32),
        jax.ShapeDtypeStruct((2, N_H, HB), jnp.float32),
    ),
    mesh=plsc.VectorSubcoreMesh(core_axis_name="c", subcore_axis_name="s"),
    compiler_params=pltpu.CompilerParams(
        needs_layout_passes=False, use_tc_tiling_on_sc=False
    ),
    scratch_types=[
        pltpu.VMEM_SHARED((N_H, HB), jnp.float32),
        pltpu.VMEM_SHARED((NT, CAP3), jnp.int32),
        pltpu.VMEM_SHARED((NT * 8,), jnp.int32),
        pltpu.VMEM((CHUNKS, C), jnp.int32),
        pltpu.VMEM((CHUNKS, C), jnp.int32),
        pltpu.VMEM((CHUNKS, C), jnp.float32),
        pltpu.VMEM((CAP3,), jnp.int32),
        pltpu.VMEM((CAP3,), jnp.int32),
        pltpu.VMEM((CAP3,), jnp.float32),
        pltpu.VMEM((CAP2,), jnp.int32),
        pltpu.VMEM((CAP2,), jnp.int32),
        pltpu.VMEM((CAP2,), jnp.float32),
        pltpu.VMEM((N_H,), jnp.float32),
        pltpu.VMEM((CAP3,), jnp.int32),
        pltpu.VMEM((LANES,), jnp.int32),
        pltpu.VMEM((NT * 8,), jnp.int32),
        pltpu.VMEM((LANES, HB), jnp.float32),
        pltpu.VMEM((C, HB), jnp.float32),
        pltpu.VMEM((C, HB), jnp.float32),
        pltpu.SemaphoreType.DMA,
        pltpu.SemaphoreType.DMA,
        pltpu.SemaphoreType.DMA,
        pltpu.SemaphoreType.DMA,
    ],
)(_body)


@jax.jit
def kernel(inp, edge_indices, edge_values):
    src = edge_indices[0].astype(jnp.int32)
    dst = edge_indices[1].astype(jnp.int32)
    val = edge_values.astype(jnp.float32)
    pad = E_PAD - E
    src = jnp.pad(src, (0, pad)).reshape(CHUNKS, NT, C)
    dst = jnp.pad(dst, (0, pad)).reshape(CHUNKS, NT, C)
    val = jnp.pad(val, (0, pad)).reshape(CHUNKS, NT, C)
    inps = inp.T.reshape(N_IN, 2, HB).transpose(1, 0, 2).reshape(
        2, N_IN * HB)
    zer = jnp.zeros((ROWS_PER_TILE, HB), jnp.float32)
    out, _, _ = _sun_sc(inps, src, dst, val, zer)
    return jnp.concatenate([out[0], out[1]], axis=1).T


# speedup vs baseline: 1.5392x; 1.0588x over previous
import functools

import jax
import jax.numpy as jnp
from jax import lax
from jax.experimental import pallas as pl
from jax.experimental.pallas import tpu as pltpu
from jax.experimental.pallas import tpu_sc as plsc

N_H = 16384
B = 64
HB = B // 2
N_IN = 512
N_OUT = 256
E = 268435
NT = 16
C = 128
CHUNKS = 136
E_PAD = NT * CHUNKS * C
CAP3 = 1152
CAP2 = 4096
ROWS_PER_TILE = N_H // NT
WB = 128
LANES = 16
GPC = C // LANES
FU = 4
JV = HB // LANES


def _splat_i32(x):
    return jnp.full((LANES,), x, jnp.int32)


def _body(inps, srcs, dsts, vals, zer, out, hs_a, hs_b,
          acc, c_src_all, cnt_all,
          src_t, dst_t, val_t, c_src, c_dst, c_val,
          c2_eid, ti0, ti1, fmap, tmp, cntb, cntb_all, rows16,
          r0, r1, g0, g1, s0, s1):
    rows = [r0, r1]
    semg = [g0, g1]
    sems = [s0, s1]
    cid = lax.axis_index("c")
    tid = lax.axis_index("s")
    zi = jnp.zeros((LANES,), jnp.int32)
    zf = jnp.zeros((LANES,), jnp.float32)
    ones16f = jnp.ones((LANES,), jnp.float32)

    _ns = jax.named_scope
    pltpu.sync_copy(srcs.at[:, tid], src_t)
    pltpu.sync_copy(dsts.at[:, tid], dst_t)
    pltpu.sync_copy(vals.at[:, tid], val_t)

    def z3_body(i, carry):
        c_src[pl.ds(i * LANES, LANES)] = zi
        c_dst[pl.ds(i * LANES, LANES)] = zi
        c_val[pl.ds(i * LANES, LANES)] = zf
        return carry

    lax.fori_loop(0, CAP3 // LANES, z3_body, 0)

    def z2_body(i, carry):
        c2_eid[pl.ds(i * LANES, LANES)] = zi
        return carry

    lax.fori_loop(0, CAP2 // LANES, z2_body, 0)

    def cp_body(g, cnt):
        c = g // GPC
        off = (g % GPC) * LANES
        d = dst_t[c, pl.ds(off, LANES)]
        m = d >= N_H - N_OUT
        npop = plsc.all_reduce_population_count(m)[0]
        ok = (cnt + npop) <= CAP3

        @pl.when(jnp.logical_and(ok, npop > 0))
        def _():
            s = src_t[c, pl.ds(off, LANES)]
            v = val_t[c, pl.ds(off, LANES)]
            plsc.store_compressed(c_dst.at[pl.ds(cnt, LANES)], d, mask=m)
            plsc.store_compressed(c_src.at[pl.ds(cnt, LANES)], s, mask=m)
            plsc.store_compressed(c_val.at[pl.ds(cnt, LANES)], v, mask=m)

        return jnp.where(ok, cnt + npop, jnp.int32(CAP3 + 1))

    with _ns("compact3"):
        n3 = lax.fori_loop(0, CHUNKS * GPC, cp_body, jnp.int32(0))

    pltpu.sync_copy(c_src, c_src_all.at[tid])
    cntb[pl.ds(0, LANES)] = _splat_i32(n3)
    pltpu.sync_copy(cntb.at[pl.ds(0, 8)], cnt_all.at[pl.ds(tid * 8, 8)])
    plsc.subcore_barrier()
    pltpu.sync_copy(cnt_all, cntb_all)
    mx = cntb_all[pl.ds(0, LANES)]
    for i in range(1, NT * 8 // LANES):
        mx = jnp.maximum(mx, cntb_all[pl.ds(i * LANES, LANES)])
    ovf = jnp.max(mx) > CAP3

    def scale(b, c, vref):
        def scale_body(i, carry2):
            if vref is val_t:
                vv = vref[c, pl.ds(i * LANES, LANES)]
            else:
                vv = vref[pl.ds(c * C + i * LANES, LANES)]
            for u in range(LANES):
                e = i * LANES + u
                vs = jnp.take_along_axis(
                    vv, jnp.full((LANES,), u, jnp.int32), axis=0)
                for j in range(JV):
                    sl = (e, pl.ds(j * LANES, LANES))
                    rows[b][sl] = rows[b][sl] * vs
            return carry2

        lax.fori_loop(0, GPC, scale_body, 0)

    ins = [None, hs_a, hs_b]
    outs = [hs_a, hs_b, None]
    iota16 = lax.iota(jnp.int32, LANES)
    for step in range(3):
        hs_in = ins[step].at[cid] if step > 0 else None

        def full_body(c, carry):
            pltpu.async_copy(hs_in.at[src_t.at[c]], rows[0], semg[0]).wait()
            scale(0, c, val_t)
            pltpu.sync_copy(rows[0], acc.at[dst_t.at[c]], add=True)
            return carry

        pltpu.sync_copy(zer, acc.at[pl.ds(tid * ROWS_PER_TILE, ROWS_PER_TILE)])
        plsc.subcore_barrier()

        if step == 0:
            pltpu.sync_copy(inps.at[cid], fmap)

            def chunk_body(c, carry):
                smin = jnp.min(src_t[c, pl.ds(0, LANES)])

                @pl.when(smin < N_IN)
                def _():
                    def grp(i, carry2):
                        svec = src_t[c, pl.ds(i * LANES, LANES)]
                        vv = val_t[c, pl.ds(i * LANES, LANES)]
                        for u in range(LANES):
                            uu = jnp.full((LANES,), u, jnp.int32)
                            sv = jnp.take_along_axis(svec, uu, axis=0)
                            vs = jnp.take_along_axis(vv, uu, axis=0)
                            okm = sv < N_IN
                            sv = jnp.where(okm, sv, 0)
                            vs = jnp.where(okm, vs, 0.0)
                            for j in range(JV):
                                addr = sv * HB + (j * LANES + iota16)
                                x = plsc.load_gather(fmap, [addr])
                                sl = (i * LANES + u, pl.ds(j * LANES, LANES))
                                rows[0][sl] = x * vs
                        return carry2

                    lax.fori_loop(0, GPC, grp, 0)
                    pltpu.sync_copy(rows[0], acc.at[dst_t.at[c]], add=True)

                return carry

            with _ns("scatter0"):
                lax.fori_loop(0, CHUNKS, chunk_body, 0)
        elif step == 1:
            with _ns("flagmap"):
                def zf_body(i, carry):
                    fmap[pl.ds(i * LANES, LANES)] = zf
                    return carry

                lax.fori_loop(0, N_H // LANES, zf_body, 0)

                @pl.when(jnp.logical_not(ovf))
                def _():
                    for r in range(NT):
                        pltpu.sync_copy(c_src_all.at[r], tmp)
                        nr = cntb_all[pl.ds(r * 8, LANES)][0]

                        def fb(g, carry):
                            svec = tmp[pl.ds(g * LANES, LANES)]
                            plsc.store_scatter(fmap, [svec], ones16f)
                            return carry

                        lax.fori_loop(0, (nr + LANES - 1) // LANES, fb, 0)

            def flush(cnt2):
                nch = (cnt2 + C - 1) // C
                tis = [ti0, ti1]

                def start_gather(b, j):
                    for u in range(GPC):
                        eidv = c2_eid[pl.ds(j * C + u * LANES, LANES)]
                        sv = plsc.load_gather(
                            src_t, [eidv >> 7, eidv & (C - 1)])
                        tis[b][pl.ds(u * LANES, LANES)] = sv
                    pltpu.async_copy(hs_in.at[tis[b]], rows[b], semg[b])

                def proc(b, j):
                    pltpu.make_async_copy(
                        hs_in.at[tis[b]], rows[b], semg[b]).wait()

                    @pl.when(j + 1 < nch)
                    def _():
                        start_gather(1 - b, j + 1)

                    for u in range(GPC):
                        gb = j * C + u * LANES
                        eidv = c2_eid[pl.ds(gb, LANES)]
                        ridx = eidv >> 7
                        cidx = eidv & (C - 1)
                        live = (_splat_i32(gb) + iota16) < cnt2
                        vv = jnp.where(
                            live, plsc.load_gather(val_t, [ridx, cidx]),
                            0.0)
                        dvec = plsc.load_gather(dst_t, [ridx, cidx])
                        for u2 in range(LANES):
                            vs = jnp.take_along_axis(
                                vv, jnp.full((LANES,), u2, jnp.int32),
                                axis=0)
                            for j2 in range(JV):
                                sl = (u * LANES + u2, pl.ds(j2 * LANES,
                                                            LANES))
                                rows[b][sl] = rows[b][sl] * vs
                        pltpu.async_copy(
                            rows[b].at[pl.ds(u * LANES, LANES)],
                            acc.at[dvec], sems[b], add=True)
                    for u in range(GPC):
                        pltpu.make_async_copy(
                            rows[b].at[pl.ds(u * LANES, LANES)],
                            acc.at[zi], sems[b]).wait()

                @pl.when(nch > 0)
                def _():
                    start_gather(0, jnp.int32(0))

                def fl_body(i, carry):
                    for b in range(2):
                        j = i * 2 + b

                        @pl.when(j < nch)
                        def _(b=b, j=j):
                            proc(b, j)

                    return carry

                lax.fori_loop(0, CAP2 // C // 2, fl_body, 0)

            @pl.when(jnp.logical_not(ovf))
            def _():
                def ft_body(q, cnt2):
                    ms_, np_ = [], []
                    for k in range(FU):
                        g = q * FU + k
                        c = g // GPC
                        off = (g % GPC) * LANES
                        d = dst_t[c, pl.ds(off, LANES)]
                        fv = plsc.load_gather(fmap, [d])
                        m = fv > 0.0
                        ms_.append(m)
                        np_.append(plsc.all_reduce_population_count(m)[0])
                    tot = (np_[0] + np_[1]) + (np_[2] + np_[3])
                    do_flush = (cnt2 + tot) > CAP2

                    @pl.when(do_flush)
                    def _():
                        flush(cnt2)

                    base = jnp.where(do_flush, jnp.int32(0), cnt2)
                    for k in range(FU):
                        g = q * FU + k
                        eidv = _splat_i32(g * LANES) + iota16
                        plsc.store_compressed(
                            c2_eid.at[pl.ds(base, LANES)], eidv,
                            mask=ms_[k])
                        base = base + np_[k]
                    return base

                with _ns("scatter1"):
                    cnt2 = lax.fori_loop(0, CHUNKS * GPC // FU, ft_body,
                                         jnp.int32(0))
                    flush(cnt2)

            @pl.when(ovf)
            def _():
                lax.fori_loop(0, CHUNKS, full_body, 0)
        else:
            @pl.when(n3 <= CAP3)
            def _():
                ng = (n3 + LANES - 1) // LANES

                def g_body(g, carry):
                    svec = c_src[pl.ds(g * LANES, LANES)]
                    dvec = c_dst[pl.ds(g * LANES, LANES)]
                    vvec = c_val[pl.ds(g * LANES, LANES)]
                    pltpu.async_copy(hs_in.at[svec], rows16, semg[0]).wait()
                    for u in range(LANES):
                        vs = jnp.take_along_axis(
                            vvec, jnp.full((LANES,), u, jnp.int32), axis=0)
                        for j in range(JV):
                            sl = (u, pl.ds(j * LANES, LANES))
                            rows16[sl] = rows16[sl] * vs
                    pltpu.sync_copy(rows16, acc.at[dvec], add=True)
                    return carry

                with _ns("scatter2"):
                    lax.fori_loop(0, ng, g_body, 0)

            @pl.when(n3 > CAP3)
            def _():
                lax.fori_loop(0, CHUNKS, full_body, 0)

        plsc.subcore_barrier()

        if step < 2:
            hs_out = outs[step].at[cid]
            base = tid * ROWS_PER_TILE

            def wb_body(k, carry):
                rbase = base + k * WB
                pltpu.sync_copy(acc.at[pl.ds(rbase, WB)], rows[0])

                def relu_body(r4, carry2):
                    for u in range(4):
                        r = r4 * 4 + u
                        keep = _splat_i32(rbase + r) >= (N_H - N_OUT)
                        for j in range(JV):
                            sl = (r, pl.ds(j * LANES, LANES))
                            x = rows[0][sl]
                            rows[0][sl] = jnp.where(keep, x,
                                                    jnp.maximum(x, 0.0))
                    return carry2

                lax.fori_loop(0, WB // 4, relu_body, 0)
                pltpu.sync_copy(rows[0], hs_out.at[pl.ds(rbase, WB)])
                return carry

            with _ns("wb" + str(step)):
                lax.fori_loop(0, ROWS_PER_TILE // WB, wb_body, 0)
        else:
            @pl.when(tid == NT - 1)
            def _():
                for k in range(N_OUT // WB):
                    rbase = N_H - N_OUT + k * WB
                    pltpu.sync_copy(acc.at[pl.ds(rbase, WB)], rows[0])

                    def sig_body(r4, carry2):
                        for u in range(4):
                            r = r4 * 4 + u
                            for j in range(JV):
                                sl = (r, pl.ds(j * LANES, LANES))
                                x = rows[0][sl]
                                rows[0][sl] = 1.0 / (1.0 + jnp.exp(-x))
                        return carry2

                    lax.fori_loop(0, WB // 4, sig_body, 0)
                    pltpu.sync_copy(rows[0], out.at[cid].at[pl.ds(k * WB, WB)])


_sun_sc = functools.partial(
    pl.kernel,
    out_type=(
        jax.ShapeDtypeStruct((2, N_OUT, HB), jnp.float32),
        jax.ShapeDtypeStruct((2, N_H, HB), jnp.float32),
        jax.ShapeDtypeStruct((2, N_H, HB), jnp.float32),
    ),
    mesh=plsc.VectorSubcoreMesh(core_axis_name="c", subcore_axis_name="s"),
    compiler_params=pltpu.CompilerParams(
        needs_layout_passes=False, use_tc_tiling_on_sc=False
    ),
    scratch_types=[
        pltpu.VMEM_SHARED((N_H, HB), jnp.float32),
        pltpu.VMEM_SHARED((NT, CAP3), jnp.int32),
        pltpu.VMEM_SHARED((NT * 8 + 16,), jnp.int32),
        pltpu.VMEM((CHUNKS, C), jnp.int32),
        pltpu.VMEM((CHUNKS, C), jnp.int32),
        pltpu.VMEM((CHUNKS, C), jnp.float32),
        pltpu.VMEM((CAP3,), jnp.int32),
        pltpu.VMEM((CAP3,), jnp.int32),
        pltpu.VMEM((CAP3,), jnp.float32),
        pltpu.VMEM((CAP2,), jnp.int32),
        pltpu.VMEM((C,), jnp.int32),
        pltpu.VMEM((C,), jnp.int32),
        pltpu.VMEM((N_H,), jnp.float32),
        pltpu.VMEM((CAP3,), jnp.int32),
        pltpu.VMEM((LANES,), jnp.int32),
        pltpu.VMEM((NT * 8 + 16,), jnp.int32),
        pltpu.VMEM((LANES, HB), jnp.float32),
        pltpu.VMEM((C, HB), jnp.float32),
        pltpu.VMEM((C, HB), jnp.float32),
        pltpu.SemaphoreType.DMA,
        pltpu.SemaphoreType.DMA,
        pltpu.SemaphoreType.DMA,
        pltpu.SemaphoreType.DMA,
    ],
)(_body)


@jax.jit
def kernel(inp, edge_indices, edge_values):
    src = edge_indices[0].astype(jnp.int32)
    dst = edge_indices[1].astype(jnp.int32)
    val = edge_values.astype(jnp.float32)
    pad = E_PAD - E
    src = jnp.pad(src, (0, pad)).reshape(CHUNKS, NT, C)
    dst = jnp.pad(dst, (0, pad)).reshape(CHUNKS, NT, C)
    val = jnp.pad(val, (0, pad)).reshape(CHUNKS, NT, C)
    inps = inp.T.reshape(N_IN, 2, HB).transpose(1, 0, 2).reshape(
        2, N_IN * HB)
    zer = jnp.zeros((ROWS_PER_TILE, HB), jnp.float32)
    out, _, _ = _sun_sc(inps, src, dst, val, zer)
    return jnp.concatenate([out[0], out[1]], axis=1).T


# speedup vs baseline: 1.6340x; 1.0616x over previous
import functools

import jax
import jax.numpy as jnp
from jax import lax
from jax.experimental import pallas as pl
from jax.experimental.pallas import tpu as pltpu
from jax.experimental.pallas import tpu_sc as plsc

N_H = 16384
B = 64
HB = B // 2
N_IN = 512
N_OUT = 256
E = 268435
NT = 16
C = 128
CHUNKS = 136
E_PAD = NT * CHUNKS * C
CAP3 = 1152
CAP2 = 4096
ROWS_PER_TILE = N_H // NT
WB = 128
LANES = 16
GPC = C // LANES
FU = 8
JV = HB // LANES


def _splat_i32(x):
    return jnp.full((LANES,), x, jnp.int32)


def _body(inps, srcs, dsts, vals, zer, out, hs_a, hs_b,
          acc, c_src_all, cnt_all,
          src_t, dst_t, val_t, c_src, c_dst, c_val,
          c2_eid, ti0, ti1, fmap, tmp, cntb, cntb_all, rows16,
          r0, r1, g0, g1, s0, s1):
    rows = [r0, r1]
    semg = [g0, g1]
    sems = [s0, s1]
    cid = lax.axis_index("c")
    tid = lax.axis_index("s")
    zi = jnp.zeros((LANES,), jnp.int32)
    zf = jnp.zeros((LANES,), jnp.float32)
    ones16f = jnp.ones((LANES,), jnp.float32)

    _ns = jax.named_scope
    pltpu.sync_copy(srcs.at[:, tid], src_t)
    pltpu.sync_copy(dsts.at[:, tid], dst_t)
    pltpu.sync_copy(vals.at[:, tid], val_t)

    def z3_body(i, carry):
        c_src[pl.ds(i * LANES, LANES)] = zi
        c_dst[pl.ds(i * LANES, LANES)] = zi
        c_val[pl.ds(i * LANES, LANES)] = zf
        return carry

    lax.fori_loop(0, CAP3 // LANES, z3_body, 0)

    def z2_body(i, carry):
        c2_eid[pl.ds(i * LANES, LANES)] = zi
        return carry

    lax.fori_loop(0, CAP2 // LANES, z2_body, 0)

    def cp_body(q, cnt):
        ms_, np_, ds_ = [], [], []
        for k in range(4):
            g = q * 4 + k
            c = g // GPC
            off = (g % GPC) * LANES
            d = dst_t[c, pl.ds(off, LANES)]
            m = d >= N_H - N_OUT
            ds_.append(d)
            ms_.append(m)
            np_.append(plsc.all_reduce_population_count(m)[0])
        tot = (np_[0] + np_[1]) + (np_[2] + np_[3])
        ok = (cnt + tot) <= CAP3

        @pl.when(jnp.logical_and(ok, tot > 0))
        def _():
            base = cnt
            for k in range(4):
                g = q * 4 + k
                c = g // GPC
                off = (g % GPC) * LANES
                sarr = src_t[c, pl.ds(off, LANES)]
                varr = val_t[c, pl.ds(off, LANES)]
                plsc.store_compressed(c_dst.at[pl.ds(base, LANES)],
                                      ds_[k], mask=ms_[k])
                plsc.store_compressed(c_src.at[pl.ds(base, LANES)],
                                      sarr, mask=ms_[k])
                plsc.store_compressed(c_val.at[pl.ds(base, LANES)],
                                      varr, mask=ms_[k])
                base = base + np_[k]

        return jnp.where(ok, cnt + tot, jnp.int32(CAP3 + 1))

    with _ns("compact3"):
        n3 = lax.fori_loop(0, CHUNKS * GPC // 4, cp_body, jnp.int32(0))

    pltpu.sync_copy(c_src, c_src_all.at[tid])
    cntb[pl.ds(0, LANES)] = _splat_i32(n3)
    pltpu.sync_copy(cntb.at[pl.ds(0, 8)], cnt_all.at[pl.ds(tid * 8, 8)])
    plsc.subcore_barrier()
    pltpu.sync_copy(cnt_all, cntb_all)
    mx = cntb_all[pl.ds(0, LANES)]
    for i in range(1, NT * 8 // LANES):
        mx = jnp.maximum(mx, cntb_all[pl.ds(i * LANES, LANES)])
    ovf = jnp.max(mx) > CAP3

    def scale(b, c, vref):
        def scale_body(i, carry2):
            if vref is val_t:
                vv = vref[c, pl.ds(i * LANES, LANES)]
            else:
                vv = vref[pl.ds(c * C + i * LANES, LANES)]
            for u in range(LANES):
                e = i * LANES + u
                vs = jnp.take_along_axis(
                    vv, jnp.full((LANES,), u, jnp.int32), axis=0)
                for j in range(JV):
                    sl = (e, pl.ds(j * LANES, LANES))
                    rows[b][sl] = rows[b][sl] * vs
            return carry2

        lax.fori_loop(0, GPC, scale_body, 0)

    ins = [None, hs_a, hs_b]
    outs = [hs_a, hs_b, None]
    iota16 = lax.iota(jnp.int32, LANES)
    for step in range(3):
        hs_in = ins[step].at[cid] if step > 0 else None

        def full_body(c, carry):
            pltpu.async_copy(hs_in.at[src_t.at[c]], rows[0], semg[0]).wait()
            scale(0, c, val_t)
            pltpu.sync_copy(rows[0], acc.at[dst_t.at[c]], add=True)
            return carry

        pltpu.sync_copy(zer, acc.at[pl.ds(tid * ROWS_PER_TILE, ROWS_PER_TILE)])
        plsc.subcore_barrier()

        if step == 0:
            pltpu.sync_copy(inps.at[cid], fmap)

            def chunk_body(c, carry):
                smin = jnp.min(src_t[c, pl.ds(0, LANES)])

                @pl.when(smin < N_IN)
                def _():
                    def grp(i, carry2):
                        svec = src_t[c, pl.ds(i * LANES, LANES)]
                        vv = val_t[c, pl.ds(i * LANES, LANES)]
                        for u in range(LANES):
                            uu = jnp.full((LANES,), u, jnp.int32)
                            sv = jnp.take_along_axis(svec, uu, axis=0)
                            vs = jnp.take_along_axis(vv, uu, axis=0)
                            okm = sv < N_IN
                            sv = jnp.where(okm, sv, 0)
                            vs = jnp.where(okm, vs, 0.0)
                            for j in range(JV):
                                addr = sv * HB + (j * LANES + iota16)
                                x = plsc.load_gather(fmap, [addr])
                                sl = (i * LANES + u, pl.ds(j * LANES, LANES))
                                rows[0][sl] = x * vs
                        return carry2

                    lax.fori_loop(0, GPC, grp, 0)
                    pltpu.sync_copy(rows[0], acc.at[dst_t.at[c]], add=True)

                return carry

            with _ns("scatter0"):
                lax.fori_loop(0, CHUNKS, chunk_body, 0)
        elif step == 1:
            with _ns("flagmap"):
                def zf_body(i, carry):
                    fmap[pl.ds(i * LANES, LANES)] = zf
                    return carry

                lax.fori_loop(0, N_H // LANES, zf_body, 0)

                @pl.when(jnp.logical_not(ovf))
                def _():
                    for r in range(NT):
                        pltpu.sync_copy(c_src_all.at[r], tmp)
                        nr = cntb_all[pl.ds(r * 8, LANES)][0]

                        def fb(g, carry):
                            svec = tmp[pl.ds(g * LANES, LANES)]
                            plsc.store_scatter(fmap, [svec], ones16f)
                            return carry

                        lax.fori_loop(0, (nr + LANES - 1) // LANES, fb, 0)

            def flush(cnt2):
                nch = (cnt2 + C - 1) // C
                tis = [ti0, ti1]

                def start_gather(b, j):
                    for u in range(GPC):
                        eidv = c2_eid[pl.ds(j * C + u * LANES, LANES)]
                        sv = plsc.load_gather(
                            src_t, [eidv >> 7, eidv & (C - 1)])
                        tis[b][pl.ds(u * LANES, LANES)] = sv
                    pltpu.async_copy(hs_in.at[tis[b]], rows[b], semg[b])

                def proc(b, j):
                    pltpu.make_async_copy(
                        hs_in.at[tis[b]], rows[b], semg[b]).wait()

                    @pl.when(j + 1 < nch)
                    def _():
                        start_gather(1 - b, j + 1)

                    for u in range(GPC):
                        gb = j * C + u * LANES
                        eidv = c2_eid[pl.ds(gb, LANES)]
                        ridx = eidv >> 7
                        cidx = eidv & (C - 1)
                        live = (_splat_i32(gb) + iota16) < cnt2
                        vv = jnp.where(
                            live, plsc.load_gather(val_t, [ridx, cidx]),
                            0.0)
                        dvec = plsc.load_gather(dst_t, [ridx, cidx])
                        for u2 in range(LANES):
                            vs = jnp.take_along_axis(
                                vv, jnp.full((LANES,), u2, jnp.int32),
                                axis=0)
                            for j2 in range(JV):
                                sl = (u * LANES + u2, pl.ds(j2 * LANES,
                                                            LANES))
                                rows[b][sl] = rows[b][sl] * vs
                        pltpu.async_copy(
                            rows[b].at[pl.ds(u * LANES, LANES)],
                            acc.at[dvec], sems[b], add=True)
                    for u in range(GPC):
                        pltpu.make_async_copy(
                            rows[b].at[pl.ds(u * LANES, LANES)],
                            acc.at[zi], sems[b]).wait()

                @pl.when(nch > 0)
                def _():
                    start_gather(0, jnp.int32(0))

                def fl_body(i, carry):
                    for b in range(2):
                        j = i * 2 + b

                        @pl.when(j < nch)
                        def _(b=b, j=j):
                            proc(b, j)

                    return carry

                lax.fori_loop(0, CAP2 // C // 2, fl_body, 0)

            @pl.when(jnp.logical_not(ovf))
            def _():
                def ft_body(q, cnt2):
                    ms_, np_ = [], []
                    for k in range(FU):
                        g = q * FU + k
                        c = g // GPC
                        off = (g % GPC) * LANES
                        d = dst_t[c, pl.ds(off, LANES)]
                        fv = plsc.load_gather(fmap, [d])
                        m = fv > 0.0
                        ms_.append(m)
                        np_.append(plsc.all_reduce_population_count(m)[0])
                    tot = np_[0]
                    for k in range(1, FU):
                        tot = tot + np_[k]
                    do_flush = (cnt2 + tot) > CAP2

                    @pl.when(do_flush)
                    def _():
                        flush(cnt2)

                    base = jnp.where(do_flush, jnp.int32(0), cnt2)
                    for k in range(FU):
                        g = q * FU + k
                        eidv = _splat_i32(g * LANES) + iota16
                        plsc.store_compressed(
                            c2_eid.at[pl.ds(base, LANES)], eidv,
                            mask=ms_[k])
                        base = base + np_[k]
                    return base

                with _ns("scatter1"):
                    cnt2 = lax.fori_loop(0, CHUNKS * GPC // FU, ft_body,
                                         jnp.int32(0))
                    flush(cnt2)

            @pl.when(ovf)
            def _():
                lax.fori_loop(0, CHUNKS, full_body, 0)
        else:
            @pl.when(n3 <= CAP3)
            def _():
                ng = (n3 + LANES - 1) // LANES

                def g_body(g, carry):
                    svec = c_src[pl.ds(g * LANES, LANES)]
                    dvec = c_dst[pl.ds(g * LANES, LANES)]
                    vvec = c_val[pl.ds(g * LANES, LANES)]
                    pltpu.async_copy(hs_in.at[svec], rows16, semg[0]).wait()
                    for u in range(LANES):
                        vs = jnp.take_along_axis(
                            vvec, jnp.full((LANES,), u, jnp.int32), axis=0)
                        for j in range(JV):
                            sl = (u, pl.ds(j * LANES, LANES))
                            rows16[sl] = rows16[sl] * vs
                    pltpu.sync_copy(rows16, acc.at[dvec], add=True)
                    return carry

                with _ns("scatter2"):
                    lax.fori_loop(0, ng, g_body, 0)

            @pl.when(n3 > CAP3)
            def _():
                lax.fori_loop(0, CHUNKS, full_body, 0)

        plsc.subcore_barrier()

        if step < 2:
            hs_out = outs[step].at[cid]
            base = tid * ROWS_PER_TILE

            def wb_body(k, carry):
                rbase = base + k * WB
                pltpu.sync_copy(acc.at[pl.ds(rbase, WB)], rows[0])

                def relu_body(r4, carry2):
                    for u in range(4):
                        r = r4 * 4 + u
                        keep = _splat_i32(rbase + r) >= (N_H - N_OUT)
                        for j in range(JV):
                            sl = (r, pl.ds(j * LANES, LANES))
                            x = rows[0][sl]
                            rows[0][sl] = jnp.where(keep, x,
                                                    jnp.maximum(x, 0.0))
                    return carry2

                lax.fori_loop(0, WB // 4, relu_body, 0)
                pltpu.sync_copy(rows[0], hs_out.at[pl.ds(rbase, WB)])
                return carry

            with _ns("wb" + str(step)):
                lax.fori_loop(0, ROWS_PER_TILE // WB, wb_body, 0)
        else:
            @pl.when(tid == NT - 1)
            def _():
                for k in range(N_OUT // WB):
                    rbase = N_H - N_OUT + k * WB
                    pltpu.sync_copy(acc.at[pl.ds(rbase, WB)], rows[0])

                    def sig_body(r4, carry2):
                        for u in range(4):
                            r = r4 * 4 + u
                            for j in range(JV):
                                sl = (r, pl.ds(j * LANES, LANES))
                                x = rows[0][sl]
                                rows[0][sl] = 1.0 / (1.0 + jnp.exp(-x))
                        return carry2

                    lax.fori_loop(0, WB // 4, sig_body, 0)
                    pltpu.sync_copy(rows[0], out.at[cid].at[pl.ds(k * WB, WB)])


_sun_sc = functools.partial(
    pl.kernel,
    out_type=(
        jax.ShapeDtypeStruct((2, N_OUT, HB), jnp.float32),
        jax.ShapeDtypeStruct((2, N_H, HB), jnp.float32),
        jax.ShapeDtypeStruct((2, N_H, HB), jnp.float32),
    ),
    mesh=plsc.VectorSubcoreMesh(core_axis_name="c", subcore_axis_name="s"),
    compiler_params=pltpu.CompilerParams(
        needs_layout_passes=False, use_tc_tiling_on_sc=False
    ),
    scratch_types=[
        pltpu.VMEM_SHARED((N_H, HB), jnp.float32),
        pltpu.VMEM_SHARED((NT, CAP3), jnp.int32),
        pltpu.VMEM_SHARED((NT * 8 + 16,), jnp.int32),
        pltpu.VMEM((CHUNKS, C), jnp.int32),
        pltpu.VMEM((CHUNKS, C), jnp.int32),
        pltpu.VMEM((CHUNKS, C), jnp.float32),
        pltpu.VMEM((CAP3,), jnp.int32),
        pltpu.VMEM((CAP3,), jnp.int32),
        pltpu.VMEM((CAP3,), jnp.float32),
        pltpu.VMEM((CAP2,), jnp.int32),
        pltpu.VMEM((C,), jnp.int32),
        pltpu.VMEM((C,), jnp.int32),
        pltpu.VMEM((N_H,), jnp.float32),
        pltpu.VMEM((CAP3,), jnp.int32),
        pltpu.VMEM((LANES,), jnp.int32),
        pltpu.VMEM((NT * 8 + 16,), jnp.int32),
        pltpu.VMEM((LANES, HB), jnp.float32),
        pltpu.VMEM((C, HB), jnp.float32),
        pltpu.VMEM((C, HB), jnp.float32),
        pltpu.SemaphoreType.DMA,
        pltpu.SemaphoreType.DMA,
        pltpu.SemaphoreType.DMA,
        pltpu.SemaphoreType.DMA,
    ],
)(_body)


@jax.jit
def kernel(inp, edge_indices, edge_values):
    src = edge_indices[0].astype(jnp.int32)
    dst = edge_indices[1].astype(jnp.int32)
    val = edge_values.astype(jnp.float32)
    pad = E_PAD - E
    src = jnp.pad(src, (0, pad)).reshape(CHUNKS, NT, C)
    dst = jnp.pad(dst, (0, pad)).reshape(CHUNKS, NT, C)
    val = jnp.pad(val, (0, pad)).reshape(CHUNKS, NT, C)
    inps = inp.T.reshape(N_IN, 2, HB).transpose(1, 0, 2).reshape(
        2, N_IN * HB)
    zer = jnp.zeros((ROWS_PER_TILE, HB), jnp.float32)
    out, _, _ = _sun_sc(inps, src, dst, val, zer)
    return jnp.concatenate([out[0], out[1]], axis=1).T


# speedup vs baseline: 1.6361x; 1.0013x over previous
import contextlib
import functools

import jax
import jax.numpy as jnp
from jax import lax
from jax.experimental import pallas as pl
from jax.experimental.pallas import tpu as pltpu
from jax.experimental.pallas import tpu_sc as plsc

N_H = 16384
B = 64
HB = B // 2
N_IN = 512
N_OUT = 256
E = 268435
NT = 16
C = 128
CHUNKS = 136
E_PAD = NT * CHUNKS * C
CAP3 = 1152
CAP2 = 4096
ROWS_PER_TILE = N_H // NT
WB = 128
LANES = 16
GPC = C // LANES
FU = 8
JV = HB // LANES


def _splat_i32(x):
    return jnp.full((LANES,), x, jnp.int32)


def _body(inps, srcs, dsts, vals, zer, out, hs_a, hs_b,
          acc, c_src_all, cnt_all,
          src_t, dst_t, val_t, c_src, c_dst, c_val,
          c2_eid, ti0, ti1, fmap, tmp, cntb, cntb_all, rows16,
          r0, r1, g0, g1, s0, s1):
    rows = [r0, r1]
    semg = [g0, g1]
    sems = [s0, s1]
    cid = lax.axis_index("c")
    tid = lax.axis_index("s")
    zi = jnp.zeros((LANES,), jnp.int32)
    zf = jnp.zeros((LANES,), jnp.float32)
    ones16f = jnp.ones((LANES,), jnp.float32)

    _ns = lambda name: contextlib.nullcontext()
    pltpu.sync_copy(srcs.at[:, tid], src_t)
    pltpu.sync_copy(dsts.at[:, tid], dst_t)
    pltpu.sync_copy(vals.at[:, tid], val_t)

    def z3_body(i, carry):
        c_src[pl.ds(i * LANES, LANES)] = zi
        c_dst[pl.ds(i * LANES, LANES)] = zi
        c_val[pl.ds(i * LANES, LANES)] = zf
        return carry

    lax.fori_loop(0, CAP3 // LANES, z3_body, 0)

    def z2_body(i, carry):
        c2_eid[pl.ds(i * LANES, LANES)] = zi
        return carry

    lax.fori_loop(0, CAP2 // LANES, z2_body, 0)

    def cp_body(q, cnt):
        ms_, np_, ds_ = [], [], []
        for k in range(4):
            g = q * 4 + k
            c = g // GPC
            off = (g % GPC) * LANES
            d = dst_t[c, pl.ds(off, LANES)]
            m = d >= N_H - N_OUT
            ds_.append(d)
            ms_.append(m)
            np_.append(plsc.all_reduce_population_count(m)[0])
        tot = (np_[0] + np_[1]) + (np_[2] + np_[3])
        ok = (cnt + tot) <= CAP3

        @pl.when(jnp.logical_and(ok, tot > 0))
        def _():
            base = cnt
            for k in range(4):
                g = q * 4 + k
                c = g // GPC
                off = (g % GPC) * LANES
                sarr = src_t[c, pl.ds(off, LANES)]
                varr = val_t[c, pl.ds(off, LANES)]
                plsc.store_compressed(c_dst.at[pl.ds(base, LANES)],
                                      ds_[k], mask=ms_[k])
                plsc.store_compressed(c_src.at[pl.ds(base, LANES)],
                                      sarr, mask=ms_[k])
                plsc.store_compressed(c_val.at[pl.ds(base, LANES)],
                                      varr, mask=ms_[k])
                base = base + np_[k]

        return jnp.where(ok, cnt + tot, jnp.int32(CAP3 + 1))

    with _ns("compact3"):
        n3 = lax.fori_loop(0, CHUNKS * GPC // 4, cp_body, jnp.int32(0))

    pltpu.sync_copy(c_src, c_src_all.at[tid])
    cntb[pl.ds(0, LANES)] = _splat_i32(n3)
    pltpu.sync_copy(cntb.at[pl.ds(0, 8)], cnt_all.at[pl.ds(tid * 8, 8)])
    plsc.subcore_barrier()
    pltpu.sync_copy(cnt_all, cntb_all)
    mx = cntb_all[pl.ds(0, LANES)]
    for i in range(1, NT * 8 // LANES):
        mx = jnp.maximum(mx, cntb_all[pl.ds(i * LANES, LANES)])
    ovf = jnp.max(mx) > CAP3

    def scale(b, c, vref):
        def scale_body(i, carry2):
            if vref is val_t:
                vv = vref[c, pl.ds(i * LANES, LANES)]
            else:
                vv = vref[pl.ds(c * C + i * LANES, LANES)]
            for u in range(LANES):
                e = i * LANES + u
                vs = jnp.take_along_axis(
                    vv, jnp.full((LANES,), u, jnp.int32), axis=0)
                for j in range(JV):
                    sl = (e, pl.ds(j * LANES, LANES))
                    rows[b][sl] = rows[b][sl] * vs
            return carry2

        lax.fori_loop(0, GPC, scale_body, 0)

    ins = [None, hs_a, hs_b]
    outs = [hs_a, hs_b, None]
    iota16 = lax.iota(jnp.int32, LANES)
    for step in range(3):
        hs_in = ins[step].at[cid] if step > 0 else None

        def full_body(c, carry):
            pltpu.async_copy(hs_in.at[src_t.at[c]], rows[0], semg[0]).wait()
            scale(0, c, val_t)
            pltpu.sync_copy(rows[0], acc.at[dst_t.at[c]], add=True)
            return carry

        pltpu.sync_copy(zer, acc.at[pl.ds(tid * ROWS_PER_TILE, ROWS_PER_TILE)])
        plsc.subcore_barrier()

        if step == 0:
            pltpu.sync_copy(inps.at[cid], fmap)

            def chunk_body(c, carry):
                smin = jnp.min(src_t[c, pl.ds(0, LANES)])

                @pl.when(smin < N_IN)
                def _():
                    def grp(i, carry2):
                        svec = src_t[c, pl.ds(i * LANES, LANES)]
                        vv = val_t[c, pl.ds(i * LANES, LANES)]
                        for u in range(LANES):
                            uu = jnp.full((LANES,), u, jnp.int32)
                            sv = jnp.take_along_axis(svec, uu, axis=0)
                            vs = jnp.take_along_axis(vv, uu, axis=0)
                            okm = sv < N_IN
                            sv = jnp.where(okm, sv, 0)
                            vs = jnp.where(okm, vs, 0.0)
                            for j in range(JV):
                                addr = sv * HB + (j * LANES + iota16)
                                x = plsc.load_gather(fmap, [addr])
                                sl = (i * LANES + u, pl.ds(j * LANES, LANES))
                                rows[0][sl] = x * vs
                        return carry2

                    lax.fori_loop(0, GPC, grp, 0)
                    pltpu.sync_copy(rows[0], acc.at[dst_t.at[c]], add=True)

                return carry

            with _ns("scatter0"):
                lax.fori_loop(0, CHUNKS, chunk_body, 0)
        elif step == 1:
            with _ns("flagmap"):
                def zf_body(i, carry):
                    fmap[pl.ds(i * LANES, LANES)] = zf
                    return carry

                lax.fori_loop(0, N_H // LANES, zf_body, 0)

                @pl.when(jnp.logical_not(ovf))
                def _():
                    for r in range(NT):
                        pltpu.sync_copy(c_src_all.at[r], tmp)
                        nr = cntb_all[pl.ds(r * 8, LANES)][0]

                        def fb(g, carry):
                            svec = tmp[pl.ds(g * LANES, LANES)]
                            plsc.store_scatter(fmap, [svec], ones16f)
                            return carry

                        lax.fori_loop(0, (nr + LANES - 1) // LANES, fb, 0)

            def flush(cnt2):
                nch = (cnt2 + C - 1) // C
                tis = [ti0, ti1]

                def start_gather(b, j):
                    for u in range(GPC):
                        eidv = c2_eid[pl.ds(j * C + u * LANES, LANES)]
                        sv = plsc.load_gather(
                            src_t, [eidv >> 7, eidv & (C - 1)])
                        tis[b][pl.ds(u * LANES, LANES)] = sv
                    pltpu.async_copy(hs_in.at[tis[b]], rows[b], semg[b])

                def proc(b, j):
                    pltpu.make_async_copy(
                        hs_in.at[tis[b]], rows[b], semg[b]).wait()

                    @pl.when(j + 1 < nch)
                    def _():
                        start_gather(1 - b, j + 1)

                    for u in range(GPC):
                        gb = j * C + u * LANES
                        eidv = c2_eid[pl.ds(gb, LANES)]
                        ridx = eidv >> 7
                        cidx = eidv & (C - 1)
                        live = (_splat_i32(gb) + iota16) < cnt2
                        vv = jnp.where(
                            live, plsc.load_gather(val_t, [ridx, cidx]),
                            0.0)
                        dvec = plsc.load_gather(dst_t, [ridx, cidx])
                        for u2 in range(LANES):
                            vs = jnp.take_along_axis(
                                vv, jnp.full((LANES,), u2, jnp.int32),
                                axis=0)
                            for j2 in range(JV):
                                sl = (u * LANES + u2, pl.ds(j2 * LANES,
                                                            LANES))
                                rows[b][sl] = rows[b][sl] * vs
                        pltpu.async_copy(
                            rows[b].at[pl.ds(u * LANES, LANES)],
                            acc.at[dvec], sems[b], add=True)
                    for u in range(GPC):
                        pltpu.make_async_copy(
                            rows[b].at[pl.ds(u * LANES, LANES)],
                            acc.at[zi], sems[b]).wait()

                @pl.when(nch > 0)
                def _():
                    start_gather(0, jnp.int32(0))

                def fl_body(i, carry):
                    for b in range(2):
                        j = i * 2 + b

                        @pl.when(j < nch)
                        def _(b=b, j=j):
                            proc(b, j)

                    return carry

                lax.fori_loop(0, CAP2 // C // 2, fl_body, 0)

            @pl.when(jnp.logical_not(ovf))
            def _():
                def ft_body(q, cnt2):
                    ms_, np_ = [], []
                    for k in range(FU):
                        g = q * FU + k
                        c = g // GPC
                        off = (g % GPC) * LANES
                        d = dst_t[c, pl.ds(off, LANES)]
                        fv = plsc.load_gather(fmap, [d])
                        m = fv > 0.0
                        ms_.append(m)
                        np_.append(plsc.all_reduce_population_count(m)[0])
                    tot = np_[0]
                    for k in range(1, FU):
                        tot = tot + np_[k]
                    do_flush = (cnt2 + tot) > CAP2

                    @pl.when(do_flush)
                    def _():
                        flush(cnt2)

                    base = jnp.where(do_flush, jnp.int32(0), cnt2)
                    for k in range(FU):
                        g = q * FU + k
                        eidv = _splat_i32(g * LANES) + iota16
                        plsc.store_compressed(
                            c2_eid.at[pl.ds(base, LANES)], eidv,
                            mask=ms_[k])
                        base = base + np_[k]
                    return base

                with _ns("scatter1"):
                    cnt2 = lax.fori_loop(0, CHUNKS * GPC // FU, ft_body,
                                         jnp.int32(0))
                    flush(cnt2)

            @pl.when(ovf)
            def _():
                lax.fori_loop(0, CHUNKS, full_body, 0)
        else:
            @pl.when(n3 <= CAP3)
            def _():
                ng = (n3 + LANES - 1) // LANES

                def g_body(g, carry):
                    svec = c_src[pl.ds(g * LANES, LANES)]
                    dvec = c_dst[pl.ds(g * LANES, LANES)]
                    vvec = c_val[pl.ds(g * LANES, LANES)]
                    pltpu.async_copy(hs_in.at[svec], rows16, semg[0]).wait()
                    for u in range(LANES):
                        vs = jnp.take_along_axis(
                            vvec, jnp.full((LANES,), u, jnp.int32), axis=0)
                        for j in range(JV):
                            sl = (u, pl.ds(j * LANES, LANES))
                            rows16[sl] = rows16[sl] * vs
                    pltpu.sync_copy(rows16, acc.at[dvec], add=True)
                    return carry

                with _ns("scatter2"):
                    lax.fori_loop(0, ng, g_body, 0)

            @pl.when(n3 > CAP3)
            def _():
                lax.fori_loop(0, CHUNKS, full_body, 0)

        plsc.subcore_barrier()

        if step < 2:
            hs_out = outs[step].at[cid]
            base = tid * ROWS_PER_TILE

            def wb_body(k, carry):
                rbase = base + k * WB
                pltpu.sync_copy(acc.at[pl.ds(rbase, WB)], rows[0])

                def relu_body(r4, carry2):
                    for u in range(4):
                        r = r4 * 4 + u
                        keep = _splat_i32(rbase + r) >= (N_H - N_OUT)
                        for j in range(JV):
                            sl = (r, pl.ds(j * LANES, LANES))
                            x = rows[0][sl]
                            rows[0][sl] = jnp.where(keep, x,
                                                    jnp.maximum(x, 0.0))
                    return carry2

                lax.fori_loop(0, WB // 4, relu_body, 0)
                pltpu.sync_copy(rows[0], hs_out.at[pl.ds(rbase, WB)])
                return carry

            with _ns("wb" + str(step)):
                lax.fori_loop(0, ROWS_PER_TILE // WB, wb_body, 0)
        else:
            @pl.when(tid == NT - 1)
            def _():
                for k in range(N_OUT // WB):
                    rbase = N_H - N_OUT + k * WB
                    pltpu.sync_copy(acc.at[pl.ds(rbase, WB)], rows[0])

                    def sig_body(r4, carry2):
                        for u in range(4):
                            r = r4 * 4 + u
                            for j in range(JV):
                                sl = (r, pl.ds(j * LANES, LANES))
                                x = rows[0][sl]
                                rows[0][sl] = 1.0 / (1.0 + jnp.exp(-x))
                        return carry2

                    lax.fori_loop(0, WB // 4, sig_body, 0)
                    pltpu.sync_copy(rows[0], out.at[cid].at[pl.ds(k * WB, WB)])


_sun_sc = functools.partial(
    pl.kernel,
    out_type=(
        jax.ShapeDtypeStruct((2, N_OUT, HB), jnp.float32),
        jax.ShapeDtypeStruct((2, N_H, HB), jnp.float32),
        jax.ShapeDtypeStruct((2, N_H, HB), jnp.float32),
    ),
    mesh=plsc.VectorSubcoreMesh(core_axis_name="c", subcore_axis_name="s"),
    compiler_params=pltpu.CompilerParams(
        needs_layout_passes=False, use_tc_tiling_on_sc=False
    ),
    scratch_types=[
        pltpu.VMEM_SHARED((N_H, HB), jnp.float32),
        pltpu.VMEM_SHARED((NT, CAP3), jnp.int32),
        pltpu.VMEM_SHARED((NT * 8 + 16,), jnp.int32),
        pltpu.VMEM((CHUNKS, C), jnp.int32),
        pltpu.VMEM((CHUNKS, C), jnp.int32),
        pltpu.VMEM((CHUNKS, C), jnp.float32),
        pltpu.VMEM((CAP3,), jnp.int32),
        pltpu.VMEM((CAP3,), jnp.int32),
        pltpu.VMEM((CAP3,), jnp.float32),
        pltpu.VMEM((CAP2,), jnp.int32),
        pltpu.VMEM((C,), jnp.int32),
        pltpu.VMEM((C,), jnp.int32),
        pltpu.VMEM((N_H,), jnp.float32),
        pltpu.VMEM((CAP3,), jnp.int32),
        pltpu.VMEM((LANES,), jnp.int32),
        pltpu.VMEM((NT * 8 + 16,), jnp.int32),
        pltpu.VMEM((LANES, HB), jnp.float32),
        pltpu.VMEM((C, HB), jnp.float32),
        pltpu.VMEM((C, HB), jnp.float32),
        pltpu.SemaphoreType.DMA,
        pltpu.SemaphoreType.DMA,
        pltpu.SemaphoreType.DMA,
        pltpu.SemaphoreType.DMA,
    ],
)(_body)


@jax.jit
def kernel(inp, edge_indices, edge_values):
    src = edge_indices[0].astype(jnp.int32)
    dst = edge_indices[1].astype(jnp.int32)
    val = edge_values.astype(jnp.float32)
    pad = E_PAD - E
    src = jnp.pad(src, (0, pad)).reshape(CHUNKS, NT, C)
    dst = jnp.pad(dst, (0, pad)).reshape(CHUNKS, NT, C)
    val = jnp.pad(val, (0, pad)).reshape(CHUNKS, NT, C)
    inps = inp.T.reshape(N_IN, 2, HB).transpose(1, 0, 2).reshape(
        2, N_IN * HB)
    zer = jnp.zeros((ROWS_PER_TILE, HB), jnp.float32)
    out, _, _ = _sun_sc(inps, src, dst, val, zer)
    return jnp.concatenate([out[0], out[1]], axis=1).T
